# baseline (speedup 1.0000x reference)
import functools

import jax
import jax.numpy as jnp
from jax import lax
from jax.experimental import pallas as pl
from jax.experimental.pallas import tpu as pltpu
from jax.experimental.pallas import tpu_sc as plsc

N = 10000
E = 320000
D = 128
NRBF = 20
NINT = 3
CUTOFF = 5.0
GAMMA = 10.0
ZMAX = 100
LOG2 = 0.6931471805599453

NC, NS = 2, 16
NW = NC * NS
EPW = E // NW
CH = 80
NCHUNK = EPW // CH
CW = 80
NROWCH = N // CW

BE = 2000
BN = 2000


def _ssp(x):
    return jax.nn.softplus(x) - LOG2


def _sc_gather_body(table, idx, out, idx_v, rows_v, sem):
    c = lax.axis_index("c")
    s = lax.axis_index("s")
    base = (s * NC + c) * EPW

    def body(i, carry):
        off = base + i * CH
        pltpu.sync_copy(idx.at[pl.ds(off, CH)], idx_v)
        pltpu.async_copy(table.at[idx_v], rows_v, sem).wait()
        pltpu.sync_copy(rows_v, out.at[pl.ds(off, CH)])
        return carry

    lax.fori_loop(0, NCHUNK, body, 0)


def _sc_gather(table, idx):
    dt = table.shape[1]
    mesh = plsc.VectorSubcoreMesh(core_axis_name="c", subcore_axis_name="s")
    k = pl.kernel(
        _sc_gather_body,
        out_type=jax.ShapeDtypeStruct((E, dt), jnp.float32),
        mesh=mesh,
        scratch_types=[
            pltpu.VMEM((CH,), jnp.int32),
            pltpu.VMEM((CH, dt), jnp.float32),
            pltpu.SemaphoreType.DMA,
        ],
        compiler_params=pltpu.CompilerParams(
            use_tc_tiling_on_sc=(dt % 128 == 0)),
    )
    return k(table, idx)


def _sc_scatter_body(vals, idx, zeros, out, idx_v, vals_v, acc_s):
    c = lax.axis_index("c")
    s = lax.axis_index("s")
    for t in range((NROWCH + NS - 1) // NS):
        k = s + NS * t

        @pl.when(k < NROWCH)
        def _():
            pltpu.sync_copy(zeros, acc_s.at[pl.ds(k * CW, CW)])

    plsc.subcore_barrier()
    base = (s * NC + c) * EPW

    def body(i, carry):
        off = base + i * CH
        pltpu.sync_copy(idx.at[pl.ds(off, CH)], idx_v)
        pltpu.sync_copy(vals.at[pl.ds(off, CH)], vals_v)
        pltpu.sync_copy(vals_v, acc_s.at[idx_v], add=True)
        return carry

    lax.fori_loop(0, NCHUNK, body, 0)
    plsc.subcore_barrier()
    for t in range((NROWCH + NS - 1) // NS):
        k = s + NS * t

        @pl.when(k < NROWCH)
        def _():
            pltpu.sync_copy(acc_s.at[pl.ds(k * CW, CW)],
                            out.at[c, pl.ds(k * CW, CW)])


def _sc_scatter(vals, idx):
    dt = vals.shape[1]
    zeros = jnp.zeros((CW, dt), jnp.float32)
    mesh = plsc.VectorSubcoreMesh(core_axis_name="c", subcore_axis_name="s")
    k = pl.kernel(
        _sc_scatter_body,
        out_type=jax.ShapeDtypeStruct((NC, N, dt), jnp.float32),
        mesh=mesh,
        scratch_types=[
            pltpu.VMEM((CH,), jnp.int32),
            pltpu.VMEM((CH, dt), jnp.float32),
            pltpu.VMEM_SHARED((N, dt), jnp.float32),
        ],
        compiler_params=pltpu.CompilerParams(
            use_tc_tiling_on_sc=(dt % 128 == 0)),
    )
    return k(vals, idx, zeros)


def _full(shape):
    return pl.BlockSpec(shape, lambda i: (0,) * len(shape))


def _geom_body(ri_ref, rj_ref, geo_ref):
    rij = rj_ref[:, :3] - ri_ref[:, :3]
    d = jnp.sqrt(jnp.sum(rij * rij, axis=1, keepdims=True) + 1e-12)
    fcut = 0.5 * (jnp.cos(jnp.pi * d / CUTOFF) + 1.0) * (d < CUTOFF)
    pad = jnp.zeros((geo_ref.shape[0], 3), jnp.float32)
    geo_ref[...] = jnp.concatenate([rij, d, fcut, pad], axis=1)


def _tc_geom(ri, rj):
    return pl.pallas_call(
        _geom_body,
        grid=(E // BE,),
        in_specs=[pl.BlockSpec((BE, 16), lambda i: (i, 0)),
                  pl.BlockSpec((BE, 16), lambda i: (i, 0))],
        out_specs=pl.BlockSpec((BE, 8), lambda i: (i, 0)),
        out_shape=jax.ShapeDtypeStruct((E, 8), jnp.float32),
    )(ri, rj)


def _atom0_body(z_ref, emb_ref, win_ref, x_ref, y_ref):
    z = z_ref[...]
    oh = (z == lax.broadcasted_iota(jnp.int32, (z.shape[0], ZMAX), 1))
    x = jnp.dot(oh.astype(jnp.float32), emb_ref[...],
                preferred_element_type=jnp.float32)
    x_ref[...] = x
    y_ref[...] = jnp.dot(x, win_ref[...], preferred_element_type=jnp.float32)


def _tc_atom0(z2, emb, win0):
    return pl.pallas_call(
        _atom0_body,
        grid=(N // BN,),
        in_specs=[pl.BlockSpec((BN, 1), lambda i: (i, 0)),
                  _full((ZMAX, D)), _full((D, D))],
        out_specs=[pl.BlockSpec((BN, D), lambda i: (i, 0)),
                   pl.BlockSpec((BN, D), lambda i: (i, 0))],
        out_shape=[jax.ShapeDtypeStruct((N, D), jnp.float32),
                   jax.ShapeDtypeStruct((N, D), jnp.float32)],
    )(z2, emb, win0)


def _rbf_of(d):
    mu = (CUTOFF / (NRBF - 1)) * lax.broadcasted_iota(
        jnp.int32, (1, NRBF), 1).astype(jnp.float32)
    return jnp.exp(-GAMMA * (d - mu) ** 2), mu


def _edge_fwd_body(geo_ref, xj_ref, wf1_ref, bf1_ref, wf2_ref, bf2_ref,
                   p_ref):
    geo = geo_ref[...]
    d = geo[:, 3:4]
    fcut = geo[:, 4:5]
    rbf, _ = _rbf_of(d)
    a = jnp.dot(rbf, wf1_ref[...],
                preferred_element_type=jnp.float32) + bf1_ref[...]
    f = jnp.dot(_ssp(a), wf2_ref[...],
                preferred_element_type=jnp.float32) + bf2_ref[...]
    p_ref[...] = xj_ref[...] * (f * fcut)


def _tc_edge_fwd(geo, xj, wf1, bf1, wf2, bf2):
    return pl.pallas_call(
        _edge_fwd_body,
        grid=(E // BE,),
        in_specs=[pl.BlockSpec((BE, 8), lambda i: (i, 0)),
                  pl.BlockSpec((BE, D), lambda i: (i, 0)),
                  _full((NRBF, D)), _full((1, D)), _full((D, D)),
                  _full((1, D))],
        out_specs=pl.BlockSpec((BE, D), lambda i: (i, 0)),
        out_shape=jax.ShapeDtypeStruct((E, D), jnp.float32),
    )(geo, xj, wf1, bf1, wf2, bf2)


def _atom_fwd_body(has_next, m2_ref, x_ref, w1_ref, b1_ref, w2_ref, b2_ref,
                   winn_ref, m_ref, xn_ref, yn_ref):
    m = m2_ref[0] + m2_ref[1]
    h = jnp.dot(m, w1_ref[...],
                preferred_element_type=jnp.float32) + b1_ref[...]
    v = jnp.dot(_ssp(h), w2_ref[...],
                preferred_element_type=jnp.float32) + b2_ref[...]
    xn = x_ref[...] + v
    m_ref[...] = m
    xn_ref[...] = xn
    if has_next:
        yn_ref[...] = jnp.dot(xn, winn_ref[...],
                              preferred_element_type=jnp.float32)
    else:
        yn_ref[...] = xn


def _tc_atom_fwd(m2, x, w1, b1, w2, b2, winn, has_next):
    return pl.pallas_call(
        functools.partial(_atom_fwd_body, has_next),
        grid=(N // BN,),
        in_specs=[pl.BlockSpec((NC, BN, D), lambda i: (0, i, 0)),
                  pl.BlockSpec((BN, D), lambda i: (i, 0)),
                  _full((D, D)), _full((1, D)), _full((D, D)),
                  _full((1, D)), _full((D, D))],
        out_specs=[pl.BlockSpec((BN, D), lambda i: (i, 0)),
                   pl.BlockSpec((BN, D), lambda i: (i, 0)),
                   pl.BlockSpec((BN, D), lambda i: (i, 0))],
        out_shape=[jax.ShapeDtypeStruct((N, D), jnp.float32),
                   jax.ShapeDtypeStruct((N, D), jnp.float32),
                   jax.ShapeDtypeStruct((N, D), jnp.float32)],
    )(m2, x, w1, b1, w2, b2, winn)


def _head_body(x3_ref, wa1_ref, ba1_ref, wa2r_ref, ba2_ref, wa1t_ref,
               gx_ref, e_ref):
    pi = pl.program_id(0)
    x3 = x3_ref[...]
    g = jnp.dot(x3, wa1_ref[...],
                preferred_element_type=jnp.float32) + ba1_ref[...]

    @pl.when(pi == 0)
    def _():
        e_ref[...] = jnp.zeros_like(e_ref)

    e_ref[...] += (jnp.sum(_ssp(g) * wa2r_ref[...], keepdims=True)
                   + x3.shape[0] * ba2_ref[...])
    gg = jax.nn.sigmoid(g) * wa2r_ref[...]
    gx_ref[...] = jnp.dot(gg, wa1t_ref[...],
                          preferred_element_type=jnp.float32)


def _tc_head(x3, wa1, ba1, wa2r, ba2, wa1t):
    return pl.pallas_call(
        _head_body,
        grid=(N // BN,),
        in_specs=[pl.BlockSpec((BN, D), lambda i: (i, 0)),
                  _full((D, D // 2)), _full((1, D // 2)),
                  _full((1, D // 2)), _full((1, 1)),
                  _full((D // 2, D))],
        out_specs=[pl.BlockSpec((BN, D), lambda i: (i, 0)),
                   pl.BlockSpec((1, 1), lambda i: (0, 0))],
        out_shape=[jax.ShapeDtypeStruct((N, D), jnp.float32),
                   jax.ShapeDtypeStruct((1, 1), jnp.float32)],
    )(x3, wa1, ba1, wa2r, ba2, wa1t)


def _atom_bwd_body(gx_ref, m_ref, w1_ref, b1_ref, w2t_ref, w1t_ref, gm_ref):
    h = jnp.dot(m_ref[...], w1_ref[...],
                preferred_element_type=jnp.float32) + b1_ref[...]
    gh = jnp.dot(gx_ref[...], w2t_ref[...],
                 preferred_element_type=jnp.float32) * jax.nn.sigmoid(h)
    gm_ref[...] = jnp.dot(gh, w1t_ref[...],
                          preferred_element_type=jnp.float32)


def _tc_atom_bwd(gx, m, w1, b1, w2t, w1t):
    return pl.pallas_call(
        _atom_bwd_body,
        grid=(N // BN,),
        in_specs=[pl.BlockSpec((BN, D), lambda i: (i, 0)),
                  pl.BlockSpec((BN, D), lambda i: (i, 0)),
                  _full((D, D)), _full((1, D)), _full((D, D)),
                  _full((D, D))],
        out_specs=pl.BlockSpec((BN, D), lambda i: (i, 0)),
        out_shape=jax.ShapeDtypeStruct((N, D), jnp.float32),
    )(gx, m, w1, b1, w2t, w1t)


def _atom_acc_body(gx_ref, gy2_ref, wint_ref, gxn_ref):
    gy = gy2_ref[0] + gy2_ref[1]
    gxn_ref[...] = gx_ref[...] + jnp.dot(
        gy, wint_ref[...], preferred_element_type=jnp.float32)


def _tc_atom_acc(gx, gy2, wint):
    return pl.pallas_call(
        _atom_acc_body,
        grid=(N // BN,),
        in_specs=[pl.BlockSpec((BN, D), lambda i: (i, 0)),
                  pl.BlockSpec((NC, BN, D), lambda i: (0, i, 0)),
                  _full((D, D))],
        out_specs=pl.BlockSpec((BN, D), lambda i: (i, 0)),
        out_shape=jax.ShapeDtypeStruct((N, D), jnp.float32),
    )(gx, gy2, wint)


def _edge_bwd_body(geo_ref, xj_ref, ge_ref, gdin_ref, wf1_ref, bf1_ref,
                   wf2_ref, bf2_ref, wf2t_ref, wf1t_ref, gxj_ref, gd_ref):
    geo = geo_ref[...]
    d = geo[:, 3:4]
    fcut = geo[:, 4:5]
    rbf, mu = _rbf_of(d)
    a = jnp.dot(rbf, wf1_ref[...],
                preferred_element_type=jnp.float32) + bf1_ref[...]
    f = jnp.dot(_ssp(a), wf2_ref[...],
                preferred_element_type=jnp.float32) + bf2_ref[...]
    ge = ge_ref[...]
    gxj_ref[...] = ge * (f * fcut)
    gw = ge * xj_ref[...]
    gf = gw * fcut
    gfc = jnp.sum(gw * f, axis=1, keepdims=True)
    ga = jnp.dot(gf, wf2t_ref[...],
                 preferred_element_type=jnp.float32) * jax.nn.sigmoid(a)
    grbf = jnp.dot(ga, wf1t_ref[...], preferred_element_type=jnp.float32)
    gd_rbf = jnp.sum(grbf * (-2.0 * GAMMA) * (d - mu) * rbf,
                     axis=1, keepdims=True)
    dfcut = (-0.5 * jnp.pi / CUTOFF) * jnp.sin(
        jnp.pi * d / CUTOFF) * (d < CUTOFF)
    gd_ref[...] = gdin_ref[...] + gd_rbf + gfc * dfcut


def _tc_edge_bwd(geo, xj, ge, gdin, wf1, bf1, wf2, bf2, wf2t, wf1t):
    return pl.pallas_call(
        _edge_bwd_body,
        grid=(E // BE,),
        in_specs=[pl.BlockSpec((BE, 8), lambda i: (i, 0)),
                  pl.BlockSpec((BE, D), lambda i: (i, 0)),
                  pl.BlockSpec((BE, D), lambda i: (i, 0)),
                  pl.BlockSpec((BE, 1), lambda i: (i, 0)),
                  _full((NRBF, D)), _full((1, D)), _full((D, D)),
                  _full((1, D)), _full((D, D)), _full((D, NRBF))],
        out_specs=[pl.BlockSpec((BE, D), lambda i: (i, 0)),
                   pl.BlockSpec((BE, 1), lambda i: (i, 0))],
        out_shape=[jax.ShapeDtypeStruct((E, D), jnp.float32),
                   jax.ShapeDtypeStruct((E, 1), jnp.float32)],
    )(geo, xj, ge, gdin, wf1, bf1, wf2, bf2, wf2t, wf1t)


def _edge_final_body(geo_ref, gd_ref, grij_ref):
    geo = geo_ref[...]
    rij = geo[:, :3]
    d = geo[:, 3:4]
    s = gd_ref[...] / d
    pad = jnp.zeros((geo.shape[0], 5), jnp.float32)
    grij_ref[...] = jnp.concatenate([s * rij, pad], axis=1)


def _tc_edge_final(geo, gd):
    return pl.pallas_call(
        _edge_final_body,
        grid=(E // BE,),
        in_specs=[pl.BlockSpec((BE, 8), lambda i: (i, 0)),
                  pl.BlockSpec((BE, 1), lambda i: (i, 0))],
        out_specs=pl.BlockSpec((BE, 8), lambda i: (i, 0)),
        out_shape=jax.ShapeDtypeStruct((E, 8), jnp.float32),
    )(geo, gd)


def _combine_body(gi_ref, gj_ref, act_ref):
    g = gi_ref[0] + gi_ref[1] - gj_ref[0] - gj_ref[1]
    act_ref[...] = g[:, :3]


def _tc_combine(gi2, gj2):
    return pl.pallas_call(
        _combine_body,
        grid=(N // BN,),
        in_specs=[pl.BlockSpec((NC, BN, 8), lambda i: (0, i, 0)),
                  pl.BlockSpec((NC, BN, 8), lambda i: (0, i, 0))],
        out_specs=pl.BlockSpec((BN, 3), lambda i: (i, 0)),
        out_shape=jax.ShapeDtypeStruct((N, 3), jnp.float32),
    )(gi2, gj2)


def kernel(R, Z, idx_i, idx_j, emb, Wf1, bf1, Wf2, bf2, Win, Wout1, bout1,
           Wout2, bout2, Wa1, ba1, Wa2, ba2):
    idx_i = idx_i.astype(jnp.int32)
    idx_j = idx_j.astype(jnp.int32)
    z2 = Z.astype(jnp.int32).reshape(N, 1)
    rt = jnp.zeros((N, 16), jnp.float32).at[:, :3].set(R)

    bf1r = bf1.reshape(NINT, 1, D)
    bf2r = bf2.reshape(NINT, 1, D)
    bo1r = bout1.reshape(NINT, 1, D)
    bo2r = bout2.reshape(NINT, 1, D)
    ba1r = ba1.reshape(1, D // 2)
    ba2r = ba2.reshape(1, 1)
    wa2r = Wa2.reshape(1, D // 2)
    wa1t = jnp.transpose(Wa1)
    wf2t = jnp.transpose(Wf2, (0, 2, 1))
    wf1t = jnp.transpose(Wf1, (0, 2, 1))
    wo1t = jnp.transpose(Wout1, (0, 2, 1))
    wo2t = jnp.transpose(Wout2, (0, 2, 1))
    wint = jnp.transpose(Win, (0, 2, 1))

    ri = _sc_gather(rt, idx_i)
    rj = _sc_gather(rt, idx_j)
    geo = _tc_geom(ri, rj)

    x, y = _tc_atom0(z2, emb, Win[0])
    ms, xjs = [], []
    for b in range(NINT):
        xj = _sc_gather(y, idx_j)
        p = _tc_edge_fwd(geo, xj, Wf1[b], bf1r[b], Wf2[b], bf2r[b])
        m2 = _sc_scatter(p, idx_i)
        winn = Win[b + 1] if b + 1 < NINT else Win[0]
        m, x, y = _tc_atom_fwd(m2, x, Wout1[b], bo1r[b], Wout2[b], bo2r[b],
                               winn, b + 1 < NINT)
        ms.append(m)
        xjs.append(xj)

    gx, e = _tc_head(x, Wa1, ba1r, wa2r, ba2r, wa1t)
    gd = jnp.zeros((E, 1), jnp.float32)
    for b in reversed(range(NINT)):
        gm = _tc_atom_bwd(gx, ms[b], Wout1[b], bo1r[b], wo2t[b], wo1t[b])
        ge = _sc_gather(gm, idx_i)
        gxj, gd = _tc_edge_bwd(geo, xjs[b], ge, gd, Wf1[b], bf1r[b],
                               Wf2[b], bf2r[b], wf2t[b], wf1t[b])
        gy2 = _sc_scatter(gxj, idx_j)
        gx = _tc_atom_acc(gx, gy2, wint[b])

    grij = _tc_edge_final(geo, gd)
    gi2 = _sc_scatter(grij, idx_i)
    gj2 = _sc_scatter(grij, idx_j)
    action = _tc_combine(gi2, gj2)
    return (action, e[0, 0])


# speedup vs baseline: 1.1981x; 1.1981x over previous
import functools

import jax
import jax.numpy as jnp
from jax import lax
from jax.experimental import pallas as pl
from jax.experimental.pallas import tpu as pltpu
from jax.experimental.pallas import tpu_sc as plsc

N = 10000
E = 320000
D = 128
NRBF = 20
NINT = 3
CUTOFF = 5.0
GAMMA = 10.0
ZMAX = 100
LOG2 = 0.6931471805599453

NC, NS = 2, 16
NW = NC * NS
EPW = E // NW
CH = 40
NCHUNK = EPW // CH
CW = 80
NROWCH = N // CW

BE = 2000
BN = 2000


def _ssp(x):
    return jax.nn.softplus(x) - LOG2


GNBUF = 5
SNBUF = 2


def _gather_pipeline(table, idx_v, out, base, rows, sem_g, sem_o):
    NBUF = GNBUF
    ROUNDS = NCHUNK // NBUF
    for b in range(NBUF):
        pltpu.async_copy(table.at[idx_v.at[b]], rows.at[b], sem_g[b])

    def rnd(r, carry):
        for b in range(NBUF):
            g = r * NBUF + b
            off = base + g * CH
            pltpu.make_async_copy(
                table.at[idx_v.at[g]], rows.at[b], sem_g[b]).wait()
            pltpu.async_copy(rows.at[b], out.at[pl.ds(off, CH)], sem_o[b])
        for b in range(NBUF):
            g = r * NBUF + b
            g2 = g + NBUF

            @pl.when(g2 < NCHUNK)
            def _():
                pltpu.make_async_copy(
                    rows.at[b], out.at[pl.ds(base + g * CH, CH)],
                    sem_o[b]).wait()
                pltpu.async_copy(table.at[idx_v.at[g2]], rows.at[b],
                                 sem_g[b])
        return carry

    lax.fori_loop(0, ROUNDS, rnd, 0)
    for b in range(NBUF):
        g = (ROUNDS - 1) * NBUF + b
        pltpu.make_async_copy(
            rows.at[b], out.at[pl.ds(base + g * CH, CH)], sem_o[b]).wait()


def _sc_gather_body(table, idx3, out, idx_v, rows, *sems):
    c = lax.axis_index("c")
    s = lax.axis_index("s")
    wid = s * NC + c
    pltpu.sync_copy(idx3.at[wid], idx_v)
    _gather_pipeline(table, idx_v, out, wid * EPW, rows,
                     sems[:GNBUF], sems[GNBUF:])


def _sc_gather(table, idx3):
    dt = table.shape[1]
    mesh = plsc.VectorSubcoreMesh(core_axis_name="c", subcore_axis_name="s")
    k = pl.kernel(
        _sc_gather_body,
        out_type=jax.ShapeDtypeStruct((E, dt), jnp.float32),
        mesh=mesh,
        scratch_types=[
            pltpu.VMEM((NCHUNK, CH), jnp.int32),
            pltpu.VMEM((GNBUF, CH, dt), jnp.float32),
        ] + [pltpu.SemaphoreType.DMA] * (2 * GNBUF),
        compiler_params=pltpu.CompilerParams(
            use_tc_tiling_on_sc=(dt % 128 == 0)),
    )
    return k(table, idx3)


def _sc_gather2_body(table, idxa3, idxb3, outa, outb, idx_v, rows, *sems):
    c = lax.axis_index("c")
    s = lax.axis_index("s")
    wid = s * NC + c
    base = wid * EPW
    pltpu.sync_copy(idxa3.at[wid], idx_v)
    _gather_pipeline(table, idx_v, outa, base, rows, sems[:GNBUF],
                     sems[GNBUF:])
    pltpu.sync_copy(idxb3.at[wid], idx_v)
    _gather_pipeline(table, idx_v, outb, base, rows, sems[:GNBUF],
                     sems[GNBUF:])


def _sc_gather2(table, idxa3, idxb3):
    dt = table.shape[1]
    mesh = plsc.VectorSubcoreMesh(core_axis_name="c", subcore_axis_name="s")
    k = pl.kernel(
        _sc_gather2_body,
        out_type=[jax.ShapeDtypeStruct((E, dt), jnp.float32),
                  jax.ShapeDtypeStruct((E, dt), jnp.float32)],
        mesh=mesh,
        scratch_types=[
            pltpu.VMEM((NCHUNK, CH), jnp.int32),
            pltpu.VMEM((GNBUF, CH, dt), jnp.float32),
        ] + [pltpu.SemaphoreType.DMA] * (2 * GNBUF),
        compiler_params=pltpu.CompilerParams(
            use_tc_tiling_on_sc=(dt % 128 == 0)),
    )
    return k(table, idxa3, idxb3)


def _acc_zero(s, zeros, acc_s):
    for t in range((NROWCH + NS - 1) // NS):
        k = s + NS * t

        @pl.when(k < NROWCH)
        def _():
            pltpu.sync_copy(zeros, acc_s.at[pl.ds(k * CW, CW)])


def _acc_writeout(c, s, acc_s, out):
    for t in range((NROWCH + NS - 1) // NS):
        k = s + NS * t

        @pl.when(k < NROWCH)
        def _():
            pltpu.sync_copy(acc_s.at[pl.ds(k * CW, CW)],
                            out.at[c, pl.ds(k * CW, CW)])


def _scatter_pipeline(vals, idx_vs, accs, base, vals_v, sem_v, sem_a):
    NBUF = SNBUF
    ROUNDS = NCHUNK // NBUF
    for b in range(NBUF):
        pltpu.async_copy(vals.at[pl.ds(base + b * CH, CH)], vals_v.at[b],
                         sem_v[b])

    def rnd(r, carry):
        descs = {}
        for b in range(NBUF):
            g = r * NBUF + b
            pltpu.make_async_copy(
                vals.at[pl.ds(base + g * CH, CH)], vals_v.at[b],
                sem_v[b]).wait()
            descs[b] = [
                pltpu.async_copy(vals_v.at[b], acc.at[iv.at[g]], sem_a[b],
                                 add=True)
                for acc, iv in zip(accs, idx_vs)]
        for b in range(NBUF):
            g2 = r * NBUF + b + NBUF

            @pl.when(g2 < NCHUNK)
            def _():
                for dsc in descs[b]:
                    dsc.wait()
                pltpu.async_copy(vals.at[pl.ds(base + g2 * CH, CH)],
                                 vals_v.at[b], sem_v[b])
        return carry

    lax.fori_loop(0, ROUNDS, rnd, 0)
    for b in range(NBUF):
        g = (ROUNDS - 1) * NBUF + b
        for acc, iv in zip(accs, idx_vs):
            pltpu.make_async_copy(vals_v.at[b], acc.at[iv.at[g]],
                                  sem_a[b]).wait()


def _sc_scatter_body(vals, idx3, zeros, out, idx_v, vals_v, acc_s, *sems):
    c = lax.axis_index("c")
    s = lax.axis_index("s")
    wid = s * NC + c
    pltpu.sync_copy(idx3.at[wid], idx_v)
    _acc_zero(s, zeros, acc_s)
    plsc.subcore_barrier()
    _scatter_pipeline(vals, [idx_v], [acc_s], wid * EPW, vals_v,
                      sems[:SNBUF], sems[SNBUF:])
    plsc.subcore_barrier()
    _acc_writeout(c, s, acc_s, out)


def _sc_scatter(vals, idx3):
    dt = vals.shape[1]
    zeros = jnp.zeros((CW, dt), jnp.float32)
    mesh = plsc.VectorSubcoreMesh(core_axis_name="c", subcore_axis_name="s")
    k = pl.kernel(
        _sc_scatter_body,
        out_type=jax.ShapeDtypeStruct((NC, N, dt), jnp.float32),
        mesh=mesh,
        scratch_types=[
            pltpu.VMEM((NCHUNK, CH), jnp.int32),
            pltpu.VMEM((SNBUF, CH, dt), jnp.float32),
            pltpu.VMEM_SHARED((N, dt), jnp.float32),
        ] + [pltpu.SemaphoreType.DMA] * (2 * SNBUF),
        compiler_params=pltpu.CompilerParams(
            use_tc_tiling_on_sc=(dt % 128 == 0)),
    )
    return k(vals, idx3, zeros)


def _sc_scatter2_body(vals, idxa3, idxb3, zeros, outa, outb, idxa_v, idxb_v,
                      vals_v, acca_s, accb_s, *sems):
    c = lax.axis_index("c")
    s = lax.axis_index("s")
    wid = s * NC + c
    pltpu.sync_copy(idxa3.at[wid], idxa_v)
    pltpu.sync_copy(idxb3.at[wid], idxb_v)
    _acc_zero(s, zeros, acca_s)
    _acc_zero(s, zeros, accb_s)
    plsc.subcore_barrier()
    _scatter_pipeline(vals, [idxa_v, idxb_v], [acca_s, accb_s], wid * EPW,
                      vals_v, sems[:SNBUF], sems[SNBUF:])
    plsc.subcore_barrier()
    _acc_writeout(c, s, acca_s, outa)
    _acc_writeout(c, s, accb_s, outb)


def _sc_scatter2(vals, idxa3, idxb3):
    dt = vals.shape[1]
    zeros = jnp.zeros((CW, dt), jnp.float32)
    mesh = plsc.VectorSubcoreMesh(core_axis_name="c", subcore_axis_name="s")
    k = pl.kernel(
        _sc_scatter2_body,
        out_type=[jax.ShapeDtypeStruct((NC, N, dt), jnp.float32),
                  jax.ShapeDtypeStruct((NC, N, dt), jnp.float32)],
        mesh=mesh,
        scratch_types=[
            pltpu.VMEM((NCHUNK, CH), jnp.int32),
            pltpu.VMEM((NCHUNK, CH), jnp.int32),
            pltpu.VMEM((SNBUF, CH, dt), jnp.float32),
            pltpu.VMEM_SHARED((N, dt), jnp.float32),
            pltpu.VMEM_SHARED((N, dt), jnp.float32),
        ] + [pltpu.SemaphoreType.DMA] * (2 * SNBUF),
        compiler_params=pltpu.CompilerParams(
            use_tc_tiling_on_sc=(dt % 128 == 0)),
    )
    return k(vals, idxa3, idxb3, zeros)


def _full(shape):
    return pl.BlockSpec(shape, lambda i: (0,) * len(shape))


def _geom_body(ri_ref, rj_ref, geo_ref):
    rij = rj_ref[:, :3] - ri_ref[:, :3]
    d = jnp.sqrt(jnp.sum(rij * rij, axis=1, keepdims=True) + 1e-12)
    fcut = 0.5 * (jnp.cos(jnp.pi * d / CUTOFF) + 1.0) * (d < CUTOFF)
    pad = jnp.zeros((geo_ref.shape[0], 3), jnp.float32)
    geo_ref[...] = jnp.concatenate([rij, d, fcut, pad], axis=1)


def _tc_geom(ri, rj):
    return pl.pallas_call(
        _geom_body,
        grid=(E // BE,),
        in_specs=[pl.BlockSpec((BE, 16), lambda i: (i, 0)),
                  pl.BlockSpec((BE, 16), lambda i: (i, 0))],
        out_specs=pl.BlockSpec((BE, 8), lambda i: (i, 0)),
        out_shape=jax.ShapeDtypeStruct((E, 8), jnp.float32),
    )(ri, rj)


def _atom0_body(z_ref, emb_ref, win_ref, x_ref, y_ref):
    z = z_ref[...]
    oh = (z == lax.broadcasted_iota(jnp.int32, (z.shape[0], ZMAX), 1))
    x = jnp.dot(oh.astype(jnp.float32), emb_ref[...],
                preferred_element_type=jnp.float32)
    x_ref[...] = x
    y_ref[...] = jnp.dot(x, win_ref[...], preferred_element_type=jnp.float32)


def _tc_atom0(z2, emb, win0):
    return pl.pallas_call(
        _atom0_body,
        grid=(N // BN,),
        in_specs=[pl.BlockSpec((BN, 1), lambda i: (i, 0)),
                  _full((ZMAX, D)), _full((D, D))],
        out_specs=[pl.BlockSpec((BN, D), lambda i: (i, 0)),
                   pl.BlockSpec((BN, D), lambda i: (i, 0))],
        out_shape=[jax.ShapeDtypeStruct((N, D), jnp.float32),
                   jax.ShapeDtypeStruct((N, D), jnp.float32)],
    )(z2, emb, win0)


def _rbf_of(d):
    mu = (CUTOFF / (NRBF - 1)) * lax.broadcasted_iota(
        jnp.int32, (1, NRBF), 1).astype(jnp.float32)
    return jnp.exp(-GAMMA * (d - mu) ** 2), mu


def _edge_fwd_body(geo_ref, xj_ref, wf1_ref, bf1_ref, wf2_ref, bf2_ref,
                   p_ref):
    geo = geo_ref[...]
    d = geo[:, 3:4]
    fcut = geo[:, 4:5]
    rbf, _ = _rbf_of(d)
    a = jnp.dot(rbf, wf1_ref[...],
                preferred_element_type=jnp.float32) + bf1_ref[...]
    f = jnp.dot(_ssp(a), wf2_ref[...],
                preferred_element_type=jnp.float32) + bf2_ref[...]
    p_ref[...] = xj_ref[...] * (f * fcut)


def _tc_edge_fwd(geo, xj, wf1, bf1, wf2, bf2):
    return pl.pallas_call(
        _edge_fwd_body,
        grid=(E // BE,),
        in_specs=[pl.BlockSpec((BE, 8), lambda i: (i, 0)),
                  pl.BlockSpec((BE, D), lambda i: (i, 0)),
                  _full((NRBF, D)), _full((1, D)), _full((D, D)),
                  _full((1, D))],
        out_specs=pl.BlockSpec((BE, D), lambda i: (i, 0)),
        out_shape=jax.ShapeDtypeStruct((E, D), jnp.float32),
    )(geo, xj, wf1, bf1, wf2, bf2)


def _atom_fwd_body(has_next, m2_ref, x_ref, w1_ref, b1_ref, w2_ref, b2_ref,
                   winn_ref, m_ref, xn_ref, yn_ref):
    m = m2_ref[0] + m2_ref[1]
    h = jnp.dot(m, w1_ref[...],
                preferred_element_type=jnp.float32) + b1_ref[...]
    v = jnp.dot(_ssp(h), w2_ref[...],
                preferred_element_type=jnp.float32) + b2_ref[...]
    xn = x_ref[...] + v
    m_ref[...] = m
    xn_ref[...] = xn
    if has_next:
        yn_ref[...] = jnp.dot(xn, winn_ref[...],
                              preferred_element_type=jnp.float32)
    else:
        yn_ref[...] = xn


def _tc_atom_fwd(m2, x, w1, b1, w2, b2, winn, has_next):
    return pl.pallas_call(
        functools.partial(_atom_fwd_body, has_next),
        grid=(N // BN,),
        in_specs=[pl.BlockSpec((NC, BN, D), lambda i: (0, i, 0)),
                  pl.BlockSpec((BN, D), lambda i: (i, 0)),
                  _full((D, D)), _full((1, D)), _full((D, D)),
                  _full((1, D)), _full((D, D))],
        out_specs=[pl.BlockSpec((BN, D), lambda i: (i, 0)),
                   pl.BlockSpec((BN, D), lambda i: (i, 0)),
                   pl.BlockSpec((BN, D), lambda i: (i, 0))],
        out_shape=[jax.ShapeDtypeStruct((N, D), jnp.float32),
                   jax.ShapeDtypeStruct((N, D), jnp.float32),
                   jax.ShapeDtypeStruct((N, D), jnp.float32)],
    )(m2, x, w1, b1, w2, b2, winn)


def _head_body(x3_ref, wa1_ref, ba1_ref, wa2r_ref, ba2_ref, wa1t_ref,
               gx_ref, e_ref):
    pi = pl.program_id(0)
    x3 = x3_ref[...]
    g = jnp.dot(x3, wa1_ref[...],
                preferred_element_type=jnp.float32) + ba1_ref[...]

    @pl.when(pi == 0)
    def _():
        e_ref[...] = jnp.zeros_like(e_ref)

    e_ref[...] += (jnp.sum(_ssp(g) * wa2r_ref[...], keepdims=True)
                   + x3.shape[0] * ba2_ref[...])
    gg = jax.nn.sigmoid(g) * wa2r_ref[...]
    gx_ref[...] = jnp.dot(gg, wa1t_ref[...],
                          preferred_element_type=jnp.float32)


def _tc_head(x3, wa1, ba1, wa2r, ba2, wa1t):
    return pl.pallas_call(
        _head_body,
        grid=(N // BN,),
        in_specs=[pl.BlockSpec((BN, D), lambda i: (i, 0)),
                  _full((D, D // 2)), _full((1, D // 2)),
                  _full((1, D // 2)), _full((1, 1)),
                  _full((D // 2, D))],
        out_specs=[pl.BlockSpec((BN, D), lambda i: (i, 0)),
                   pl.BlockSpec((1, 1), lambda i: (0, 0))],
        out_shape=[jax.ShapeDtypeStruct((N, D), jnp.float32),
                   jax.ShapeDtypeStruct((1, 1), jnp.float32)],
    )(x3, wa1, ba1, wa2r, ba2, wa1t)


def _atom_bwd_body(gx_ref, m_ref, w1_ref, b1_ref, w2t_ref, w1t_ref, gm_ref):
    h = jnp.dot(m_ref[...], w1_ref[...],
                preferred_element_type=jnp.float32) + b1_ref[...]
    gh = jnp.dot(gx_ref[...], w2t_ref[...],
                 preferred_element_type=jnp.float32) * jax.nn.sigmoid(h)
    gm_ref[...] = jnp.dot(gh, w1t_ref[...],
                          preferred_element_type=jnp.float32)


def _tc_atom_bwd(gx, m, w1, b1, w2t, w1t):
    return pl.pallas_call(
        _atom_bwd_body,
        grid=(N // BN,),
        in_specs=[pl.BlockSpec((BN, D), lambda i: (i, 0)),
                  pl.BlockSpec((BN, D), lambda i: (i, 0)),
                  _full((D, D)), _full((1, D)), _full((D, D)),
                  _full((D, D))],
        out_specs=pl.BlockSpec((BN, D), lambda i: (i, 0)),
        out_shape=jax.ShapeDtypeStruct((N, D), jnp.float32),
    )(gx, m, w1, b1, w2t, w1t)


def _atom_acc_body(gx_ref, gy2_ref, wint_ref, gxn_ref):
    gy = gy2_ref[0] + gy2_ref[1]
    gxn_ref[...] = gx_ref[...] + jnp.dot(
        gy, wint_ref[...], preferred_element_type=jnp.float32)


def _tc_atom_acc(gx, gy2, wint):
    return pl.pallas_call(
        _atom_acc_body,
        grid=(N // BN,),
        in_specs=[pl.BlockSpec((BN, D), lambda i: (i, 0)),
                  pl.BlockSpec((NC, BN, D), lambda i: (0, i, 0)),
                  _full((D, D))],
        out_specs=pl.BlockSpec((BN, D), lambda i: (i, 0)),
        out_shape=jax.ShapeDtypeStruct((N, D), jnp.float32),
    )(gx, gy2, wint)


def _edge_bwd_body(geo_ref, xj_ref, ge_ref, gdin_ref, wf1_ref, bf1_ref,
                   wf2_ref, bf2_ref, wf2t_ref, wf1t_ref, gxj_ref, gd_ref):
    geo = geo_ref[...]
    d = geo[:, 3:4]
    fcut = geo[:, 4:5]
    rbf, mu = _rbf_of(d)
    a = jnp.dot(rbf, wf1_ref[...],
                preferred_element_type=jnp.float32) + bf1_ref[...]
    f = jnp.dot(_ssp(a), wf2_ref[...],
                preferred_element_type=jnp.float32) + bf2_ref[...]
    ge = ge_ref[...]
    gxj_ref[...] = ge * (f * fcut)
    gw = ge * xj_ref[...]
    gf = gw * fcut
    gfc = jnp.sum(gw * f, axis=1, keepdims=True)
    ga = jnp.dot(gf, wf2t_ref[...],
                 preferred_element_type=jnp.float32) * jax.nn.sigmoid(a)
    grbf = jnp.dot(ga, wf1t_ref[...], preferred_element_type=jnp.float32)
    gd_rbf = jnp.sum(grbf * (-2.0 * GAMMA) * (d - mu) * rbf,
                     axis=1, keepdims=True)
    dfcut = (-0.5 * jnp.pi / CUTOFF) * jnp.sin(
        jnp.pi * d / CUTOFF) * (d < CUTOFF)
    gd_ref[...] = gdin_ref[...] + gd_rbf + gfc * dfcut


def _tc_edge_bwd(geo, xj, ge, gdin, wf1, bf1, wf2, bf2, wf2t, wf1t):
    return pl.pallas_call(
        _edge_bwd_body,
        grid=(E // BE,),
        in_specs=[pl.BlockSpec((BE, 8), lambda i: (i, 0)),
                  pl.BlockSpec((BE, D), lambda i: (i, 0)),
                  pl.BlockSpec((BE, D), lambda i: (i, 0)),
                  pl.BlockSpec((BE, 1), lambda i: (i, 0)),
                  _full((NRBF, D)), _full((1, D)), _full((D, D)),
                  _full((1, D)), _full((D, D)), _full((D, NRBF))],
        out_specs=[pl.BlockSpec((BE, D), lambda i: (i, 0)),
                   pl.BlockSpec((BE, 1), lambda i: (i, 0))],
        out_shape=[jax.ShapeDtypeStruct((E, D), jnp.float32),
                   jax.ShapeDtypeStruct((E, 1), jnp.float32)],
    )(geo, xj, ge, gdin, wf1, bf1, wf2, bf2, wf2t, wf1t)


def _edge_final_body(geo_ref, gd_ref, grij_ref):
    geo = geo_ref[...]
    rij = geo[:, :3]
    d = geo[:, 3:4]
    s = gd_ref[...] / d
    pad = jnp.zeros((geo.shape[0], 5), jnp.float32)
    grij_ref[...] = jnp.concatenate([s * rij, pad], axis=1)


def _tc_edge_final(geo, gd):
    return pl.pallas_call(
        _edge_final_body,
        grid=(E // BE,),
        in_specs=[pl.BlockSpec((BE, 8), lambda i: (i, 0)),
                  pl.BlockSpec((BE, 1), lambda i: (i, 0))],
        out_specs=pl.BlockSpec((BE, 8), lambda i: (i, 0)),
        out_shape=jax.ShapeDtypeStruct((E, 8), jnp.float32),
    )(geo, gd)


def _combine_body(gi_ref, gj_ref, act_ref):
    g = gi_ref[0] + gi_ref[1] - gj_ref[0] - gj_ref[1]
    act_ref[...] = g[:, :3]


def _tc_combine(gi2, gj2):
    return pl.pallas_call(
        _combine_body,
        grid=(N // BN,),
        in_specs=[pl.BlockSpec((NC, BN, 8), lambda i: (0, i, 0)),
                  pl.BlockSpec((NC, BN, 8), lambda i: (0, i, 0))],
        out_specs=pl.BlockSpec((BN, 3), lambda i: (i, 0)),
        out_shape=jax.ShapeDtypeStruct((N, 3), jnp.float32),
    )(gi2, gj2)


def kernel(R, Z, idx_i, idx_j, emb, Wf1, bf1, Wf2, bf2, Win, Wout1, bout1,
           Wout2, bout2, Wa1, ba1, Wa2, ba2):
    idx_i3 = idx_i.astype(jnp.int32).reshape(NW, NCHUNK, CH)
    idx_j3 = idx_j.astype(jnp.int32).reshape(NW, NCHUNK, CH)
    z2 = Z.astype(jnp.int32).reshape(N, 1)
    rt = jnp.zeros((N, 16), jnp.float32).at[:, :3].set(R)

    bf1r = bf1.reshape(NINT, 1, D)
    bf2r = bf2.reshape(NINT, 1, D)
    bo1r = bout1.reshape(NINT, 1, D)
    bo2r = bout2.reshape(NINT, 1, D)
    ba1r = ba1.reshape(1, D // 2)
    ba2r = ba2.reshape(1, 1)
    wa2r = Wa2.reshape(1, D // 2)
    wa1t = jnp.transpose(Wa1)
    wf2t = jnp.transpose(Wf2, (0, 2, 1))
    wf1t = jnp.transpose(Wf1, (0, 2, 1))
    wo1t = jnp.transpose(Wout1, (0, 2, 1))
    wo2t = jnp.transpose(Wout2, (0, 2, 1))
    wint = jnp.transpose(Win, (0, 2, 1))

    ri, rj = _sc_gather2(rt, idx_i3, idx_j3)
    geo = _tc_geom(ri, rj)

    x, y = _tc_atom0(z2, emb, Win[0])
    ms, xjs = [], []
    for b in range(NINT):
        xj = _sc_gather(y, idx_j3)
        p = _tc_edge_fwd(geo, xj, Wf1[b], bf1r[b], Wf2[b], bf2r[b])
        m2 = _sc_scatter(p, idx_i3)
        winn = Win[b + 1] if b + 1 < NINT else Win[0]
        m, x, y = _tc_atom_fwd(m2, x, Wout1[b], bo1r[b], Wout2[b], bo2r[b],
                               winn, b + 1 < NINT)
        ms.append(m)
        xjs.append(xj)

    gx, e = _tc_head(x, Wa1, ba1r, wa2r, ba2r, wa1t)
    gd = jnp.zeros((E, 1), jnp.float32)
    for b in reversed(range(NINT)):
        gm = _tc_atom_bwd(gx, ms[b], Wout1[b], bo1r[b], wo2t[b], wo1t[b])
        ge = _sc_gather(gm, idx_i3)
        gxj, gd = _tc_edge_bwd(geo, xjs[b], ge, gd, Wf1[b], bf1r[b],
                               Wf2[b], bf2r[b], wf2t[b], wf1t[b])
        gy2 = _sc_scatter(gxj, idx_j3)
        gx = _tc_atom_acc(gx, gy2, wint[b])

    grij = _tc_edge_final(geo, gd)
    gi2, gj2 = _sc_scatter2(grij, idx_i3, idx_j3)
    action = _tc_combine(gi2, gj2)
    return (action, e[0, 0])


# speedup vs baseline: 1.4460x; 1.2069x over previous
import functools

import jax
import jax.numpy as jnp
from jax import lax
from jax.experimental import pallas as pl
from jax.experimental.pallas import tpu as pltpu
from jax.experimental.pallas import tpu_sc as plsc

N = 10000
E = 320000
D = 128
NRBF = 20
NINT = 3
CUTOFF = 5.0
GAMMA = 10.0
ZMAX = 100
LOG2 = 0.6931471805599453

NC, NS = 2, 16
NW = NC * NS
EPW = E // NW
CH = 40
NCHUNK = EPW // CH
CW = 80
NROWCH = N // CW

BE = 2000
BN = 2000


def _ssp(x):
    return jax.nn.softplus(x) - LOG2


GNBUF = 5
SNBUF = 2


def _gather_pipeline(table, idx_v, out, base, rows, sem_g, sem_o):
    NBUF = GNBUF
    ROUNDS = NCHUNK // NBUF
    for b in range(NBUF):
        pltpu.async_copy(table.at[idx_v.at[b]], rows.at[b], sem_g[b])

    def rnd(r, carry):
        for b in range(NBUF):
            g = r * NBUF + b
            off = base + g * CH
            pltpu.make_async_copy(
                table.at[idx_v.at[g]], rows.at[b], sem_g[b]).wait()
            pltpu.async_copy(rows.at[b], out.at[pl.ds(off, CH)], sem_o[b])
        for b in range(NBUF):
            g = r * NBUF + b
            g2 = g + NBUF

            @pl.when(g2 < NCHUNK)
            def _():
                pltpu.make_async_copy(
                    rows.at[b], out.at[pl.ds(base + g * CH, CH)],
                    sem_o[b]).wait()
                pltpu.async_copy(table.at[idx_v.at[g2]], rows.at[b],
                                 sem_g[b])
        return carry

    lax.fori_loop(0, ROUNDS, rnd, 0)
    for b in range(NBUF):
        g = (ROUNDS - 1) * NBUF + b
        pltpu.make_async_copy(
            rows.at[b], out.at[pl.ds(base + g * CH, CH)], sem_o[b]).wait()


def _sc_gather_body(table, idx3, out, idx_v, rows, *sems):
    c = lax.axis_index("c")
    s = lax.axis_index("s")
    wid = s * NC + c
    pltpu.sync_copy(idx3.at[wid], idx_v)
    _gather_pipeline(table, idx_v, out, wid * EPW, rows,
                     sems[:GNBUF], sems[GNBUF:])


def _sc_gather(table, idx3):
    dt = table.shape[1]
    mesh = plsc.VectorSubcoreMesh(core_axis_name="c", subcore_axis_name="s")
    k = pl.kernel(
        _sc_gather_body,
        out_type=jax.ShapeDtypeStruct((E, dt), jnp.float32),
        mesh=mesh,
        scratch_types=[
            pltpu.VMEM((NCHUNK, CH), jnp.int32),
            pltpu.VMEM((GNBUF, CH, dt), jnp.float32),
        ] + [pltpu.SemaphoreType.DMA] * (2 * GNBUF),
        compiler_params=pltpu.CompilerParams(
            use_tc_tiling_on_sc=(dt % 128 == 0)),
    )
    return k(table, idx3)


def _sc_gather2_body(table, idxa3, idxb3, outa, outb, idx_v, rows, *sems):
    c = lax.axis_index("c")
    s = lax.axis_index("s")
    wid = s * NC + c
    base = wid * EPW
    pltpu.sync_copy(idxa3.at[wid], idx_v)
    _gather_pipeline(table, idx_v, outa, base, rows, sems[:GNBUF],
                     sems[GNBUF:])
    pltpu.sync_copy(idxb3.at[wid], idx_v)
    _gather_pipeline(table, idx_v, outb, base, rows, sems[:GNBUF],
                     sems[GNBUF:])


def _sc_gather2(table, idxa3, idxb3):
    dt = table.shape[1]
    mesh = plsc.VectorSubcoreMesh(core_axis_name="c", subcore_axis_name="s")
    k = pl.kernel(
        _sc_gather2_body,
        out_type=[jax.ShapeDtypeStruct((E, dt), jnp.float32),
                  jax.ShapeDtypeStruct((E, dt), jnp.float32)],
        mesh=mesh,
        scratch_types=[
            pltpu.VMEM((NCHUNK, CH), jnp.int32),
            pltpu.VMEM((GNBUF, CH, dt), jnp.float32),
        ] + [pltpu.SemaphoreType.DMA] * (2 * GNBUF),
        compiler_params=pltpu.CompilerParams(
            use_tc_tiling_on_sc=(dt % 128 == 0)),
    )
    return k(table, idxa3, idxb3)


def _acc_zero(s, zeros, acc_s):
    for t in range((NROWCH + NS - 1) // NS):
        k = s + NS * t

        @pl.when(k < NROWCH)
        def _():
            pltpu.sync_copy(zeros, acc_s.at[pl.ds(k * CW, CW)])


def _acc_writeout(c, s, acc_s, out):
    for t in range((NROWCH + NS - 1) // NS):
        k = s + NS * t

        @pl.when(k < NROWCH)
        def _():
            pltpu.sync_copy(acc_s.at[pl.ds(k * CW, CW)],
                            out.at[c, pl.ds(k * CW, CW)])


def _scatter_pipeline(vals, idx_vs, accs, base, vals_v, sem_v, sem_a):
    NBUF = SNBUF
    ROUNDS = NCHUNK // NBUF
    for b in range(NBUF):
        pltpu.async_copy(vals.at[pl.ds(base + b * CH, CH)], vals_v.at[b],
                         sem_v[b])

    def rnd(r, carry):
        descs = {}
        for b in range(NBUF):
            g = r * NBUF + b
            pltpu.make_async_copy(
                vals.at[pl.ds(base + g * CH, CH)], vals_v.at[b],
                sem_v[b]).wait()
            descs[b] = [
                pltpu.async_copy(vals_v.at[b], acc.at[iv.at[g]], sem_a[b],
                                 add=True)
                for acc, iv in zip(accs, idx_vs)]
        for b in range(NBUF):
            g2 = r * NBUF + b + NBUF

            @pl.when(g2 < NCHUNK)
            def _():
                for dsc in descs[b]:
                    dsc.wait()
                pltpu.async_copy(vals.at[pl.ds(base + g2 * CH, CH)],
                                 vals_v.at[b], sem_v[b])
        return carry

    lax.fori_loop(0, ROUNDS, rnd, 0)
    for b in range(NBUF):
        g = (ROUNDS - 1) * NBUF + b
        for acc, iv in zip(accs, idx_vs):
            pltpu.make_async_copy(vals_v.at[b], acc.at[iv.at[g]],
                                  sem_a[b]).wait()


def _sc_scatter_body(vals, idx3, zeros, out, idx_v, vals_v, acc_s, *sems):
    c = lax.axis_index("c")
    s = lax.axis_index("s")
    wid = s * NC + c
    pltpu.sync_copy(idx3.at[wid], idx_v)
    _acc_zero(s, zeros, acc_s)
    plsc.subcore_barrier()
    _scatter_pipeline(vals, [idx_v], [acc_s], wid * EPW, vals_v,
                      sems[:SNBUF], sems[SNBUF:])
    plsc.subcore_barrier()
    _acc_writeout(c, s, acc_s, out)


def _sc_scatter(vals, idx3):
    dt = vals.shape[1]
    zeros = jnp.zeros((CW, dt), jnp.float32)
    mesh = plsc.VectorSubcoreMesh(core_axis_name="c", subcore_axis_name="s")
    k = pl.kernel(
        _sc_scatter_body,
        out_type=jax.ShapeDtypeStruct((NC, N, dt), jnp.float32),
        mesh=mesh,
        scratch_types=[
            pltpu.VMEM((NCHUNK, CH), jnp.int32),
            pltpu.VMEM((SNBUF, CH, dt), jnp.float32),
            pltpu.VMEM_SHARED((N, dt), jnp.float32),
        ] + [pltpu.SemaphoreType.DMA] * (2 * SNBUF),
        compiler_params=pltpu.CompilerParams(
            use_tc_tiling_on_sc=(dt % 128 == 0)),
    )
    return k(vals, idx3, zeros)


def _sc_scatter2_body(vals, idxa3, idxb3, zeros, outa, outb, idxa_v, idxb_v,
                      vals_v, acca_s, accb_s, *sems):
    c = lax.axis_index("c")
    s = lax.axis_index("s")
    wid = s * NC + c
    pltpu.sync_copy(idxa3.at[wid], idxa_v)
    pltpu.sync_copy(idxb3.at[wid], idxb_v)
    _acc_zero(s, zeros, acca_s)
    _acc_zero(s, zeros, accb_s)
    plsc.subcore_barrier()
    _scatter_pipeline(vals, [idxa_v, idxb_v], [acca_s, accb_s], wid * EPW,
                      vals_v, sems[:SNBUF], sems[SNBUF:])
    plsc.subcore_barrier()
    _acc_writeout(c, s, acca_s, outa)
    _acc_writeout(c, s, accb_s, outb)


def _sc_scatter2(vals, idxa3, idxb3):
    dt = vals.shape[1]
    zeros = jnp.zeros((CW, dt), jnp.float32)
    mesh = plsc.VectorSubcoreMesh(core_axis_name="c", subcore_axis_name="s")
    k = pl.kernel(
        _sc_scatter2_body,
        out_type=[jax.ShapeDtypeStruct((NC, N, dt), jnp.float32),
                  jax.ShapeDtypeStruct((NC, N, dt), jnp.float32)],
        mesh=mesh,
        scratch_types=[
            pltpu.VMEM((NCHUNK, CH), jnp.int32),
            pltpu.VMEM((NCHUNK, CH), jnp.int32),
            pltpu.VMEM((SNBUF, CH, dt), jnp.float32),
            pltpu.VMEM_SHARED((N, dt), jnp.float32),
            pltpu.VMEM_SHARED((N, dt), jnp.float32),
        ] + [pltpu.SemaphoreType.DMA] * (2 * SNBUF),
        compiler_params=pltpu.CompilerParams(
            use_tc_tiling_on_sc=(dt % 128 == 0)),
    )
    return k(vals, idxa3, idxb3, zeros)


def _full(shape):
    return pl.BlockSpec(shape, lambda i: (0,) * len(shape))


BEG = 2000


def _geom_body(ri_ref, rj_ref, geo_ref):
    rij = rj_ref[:, :3] - ri_ref[:, :3]
    d2 = jnp.sum(rij * rij, axis=1, keepdims=True) + 1e-12
    d = jnp.sqrt(d2)
    mask = d < CUTOFF
    ang = jnp.pi * d / CUTOFF
    fcut = 0.5 * (jnp.cos(ang) + 1.0) * mask
    dfc = (-0.5 * jnp.pi / CUTOFF) * jnp.sin(ang) * mask
    pad = jnp.zeros((BEG, 2), jnp.float32)
    geo_ref[...] = jnp.concatenate([rij, d, fcut, dfc, pad], axis=1)


def _tc_geom(ri, rj):
    return pl.pallas_call(
        _geom_body,
        grid=(E // BEG,),
        in_specs=[pl.BlockSpec((BEG, 16), lambda i: (i, 0)),
                  pl.BlockSpec((BEG, 16), lambda i: (i, 0))],
        out_specs=pl.BlockSpec((BEG, 8), lambda i: (i, 0)),
        out_shape=jax.ShapeDtypeStruct((E, 8), jnp.float32),
    )(ri, rj)


def _atom0_body(z_ref, emb_ref, win_ref, x_ref, y_ref):
    z = z_ref[...]
    oh = (z == lax.broadcasted_iota(jnp.int32, (z.shape[0], ZMAX), 1))
    x = jnp.dot(oh.astype(jnp.float32), emb_ref[...],
                preferred_element_type=jnp.float32)
    x_ref[...] = x
    y_ref[...] = jnp.dot(x, win_ref[...], preferred_element_type=jnp.float32)


def _tc_atom0(z2, emb, win0):
    return pl.pallas_call(
        _atom0_body,
        grid=(N // BN,),
        in_specs=[pl.BlockSpec((BN, 1), lambda i: (i, 0)),
                  _full((ZMAX, D)), _full((D, D))],
        out_specs=[pl.BlockSpec((BN, D), lambda i: (i, 0)),
                   pl.BlockSpec((BN, D), lambda i: (i, 0))],
        out_shape=[jax.ShapeDtypeStruct((N, D), jnp.float32),
                   jax.ShapeDtypeStruct((N, D), jnp.float32)],
    )(z2, emb, win0)


def _rbf_of(d):
    mu = (CUTOFF / (NRBF - 1)) * lax.broadcasted_iota(
        jnp.int32, (1, NRBF), 1).astype(jnp.float32)
    return jnp.exp(-GAMMA * (d - mu) ** 2), mu


def _edge_fwd_body(geo_ref, xj_ref, wf1_ref, bf1_ref, wf2_ref, bf2_ref,
                   p_ref):
    geo = geo_ref[...]
    d = geo[:, 3:4]
    fcut = geo[:, 4:5]
    rbf, _ = _rbf_of(d)
    a = jnp.dot(rbf, wf1_ref[...],
                preferred_element_type=jnp.float32) + bf1_ref[...]
    f = jnp.dot(_ssp(a), wf2_ref[...],
                preferred_element_type=jnp.float32) + bf2_ref[...]
    p_ref[...] = xj_ref[...] * (f * fcut)


def _tc_edge_fwd(geo, xj, wf1, bf1, wf2, bf2):
    return pl.pallas_call(
        _edge_fwd_body,
        grid=(E // BE,),
        in_specs=[pl.BlockSpec((BE, 8), lambda i: (i, 0)),
                  pl.BlockSpec((BE, D), lambda i: (i, 0)),
                  _full((NRBF, D)), _full((1, D)), _full((D, D)),
                  _full((1, D))],
        out_specs=pl.BlockSpec((BE, D), lambda i: (i, 0)),
        out_shape=jax.ShapeDtypeStruct((E, D), jnp.float32),
    )(geo, xj, wf1, bf1, wf2, bf2)


def _atom_fwd_body(has_next, m2_ref, x_ref, w1_ref, b1_ref, w2_ref, b2_ref,
                   winn_ref, m_ref, xn_ref, yn_ref):
    m = m2_ref[0] + m2_ref[1]
    h = jnp.dot(m, w1_ref[...],
                preferred_element_type=jnp.float32) + b1_ref[...]
    v = jnp.dot(_ssp(h), w2_ref[...],
                preferred_element_type=jnp.float32) + b2_ref[...]
    xn = x_ref[...] + v
    m_ref[...] = m
    xn_ref[...] = xn
    if has_next:
        yn_ref[...] = jnp.dot(xn, winn_ref[...],
                              preferred_element_type=jnp.float32)
    else:
        yn_ref[...] = xn


def _tc_atom_fwd(m2, x, w1, b1, w2, b2, winn, has_next):
    return pl.pallas_call(
        functools.partial(_atom_fwd_body, has_next),
        grid=(N // BN,),
        in_specs=[pl.BlockSpec((NC, BN, D), lambda i: (0, i, 0)),
                  pl.BlockSpec((BN, D), lambda i: (i, 0)),
                  _full((D, D)), _full((1, D)), _full((D, D)),
                  _full((1, D)), _full((D, D))],
        out_specs=[pl.BlockSpec((BN, D), lambda i: (i, 0)),
                   pl.BlockSpec((BN, D), lambda i: (i, 0)),
                   pl.BlockSpec((BN, D), lambda i: (i, 0))],
        out_shape=[jax.ShapeDtypeStruct((N, D), jnp.float32),
                   jax.ShapeDtypeStruct((N, D), jnp.float32),
                   jax.ShapeDtypeStruct((N, D), jnp.float32)],
    )(m2, x, w1, b1, w2, b2, winn)


def _head_body(x3_ref, wa1_ref, ba1_ref, wa2r_ref, ba2_ref, wa1t_ref,
               gx_ref, e_ref):
    pi = pl.program_id(0)
    x3 = x3_ref[...]
    g = jnp.dot(x3, wa1_ref[...],
                preferred_element_type=jnp.float32) + ba1_ref[...]

    @pl.when(pi == 0)
    def _():
        e_ref[...] = jnp.zeros_like(e_ref)

    e_ref[...] += (jnp.sum(_ssp(g) * wa2r_ref[...], keepdims=True)
                   + x3.shape[0] * ba2_ref[...])
    gg = jax.nn.sigmoid(g) * wa2r_ref[...]
    gx_ref[...] = jnp.dot(gg, wa1t_ref[...],
                          preferred_element_type=jnp.float32)


def _tc_head(x3, wa1, ba1, wa2r, ba2, wa1t):
    return pl.pallas_call(
        _head_body,
        grid=(N // BN,),
        in_specs=[pl.BlockSpec((BN, D), lambda i: (i, 0)),
                  _full((D, D // 2)), _full((1, D // 2)),
                  _full((1, D // 2)), _full((1, 1)),
                  _full((D // 2, D))],
        out_specs=[pl.BlockSpec((BN, D), lambda i: (i, 0)),
                   pl.BlockSpec((1, 1), lambda i: (0, 0))],
        out_shape=[jax.ShapeDtypeStruct((N, D), jnp.float32),
                   jax.ShapeDtypeStruct((1, 1), jnp.float32)],
    )(x3, wa1, ba1, wa2r, ba2, wa1t)


def _atom_bwd_body(gx_ref, m_ref, w1_ref, b1_ref, w2t_ref, w1t_ref, gm_ref):
    h = jnp.dot(m_ref[...], w1_ref[...],
                preferred_element_type=jnp.float32) + b1_ref[...]
    gh = jnp.dot(gx_ref[...], w2t_ref[...],
                 preferred_element_type=jnp.float32) * jax.nn.sigmoid(h)
    gm_ref[...] = jnp.dot(gh, w1t_ref[...],
                          preferred_element_type=jnp.float32)


def _tc_atom_bwd(gx, m, w1, b1, w2t, w1t):
    return pl.pallas_call(
        _atom_bwd_body,
        grid=(N // BN,),
        in_specs=[pl.BlockSpec((BN, D), lambda i: (i, 0)),
                  pl.BlockSpec((BN, D), lambda i: (i, 0)),
                  _full((D, D)), _full((1, D)), _full((D, D)),
                  _full((D, D))],
        out_specs=pl.BlockSpec((BN, D), lambda i: (i, 0)),
        out_shape=jax.ShapeDtypeStruct((N, D), jnp.float32),
    )(gx, m, w1, b1, w2t, w1t)


def _atom_acc_body(gx_ref, gy2_ref, wint_ref, gxn_ref):
    gy = gy2_ref[0] + gy2_ref[1]
    gxn_ref[...] = gx_ref[...] + jnp.dot(
        gy, wint_ref[...], preferred_element_type=jnp.float32)


def _tc_atom_acc(gx, gy2, wint):
    return pl.pallas_call(
        _atom_acc_body,
        grid=(N // BN,),
        in_specs=[pl.BlockSpec((BN, D), lambda i: (i, 0)),
                  pl.BlockSpec((NC, BN, D), lambda i: (0, i, 0)),
                  _full((D, D))],
        out_specs=pl.BlockSpec((BN, D), lambda i: (i, 0)),
        out_shape=jax.ShapeDtypeStruct((N, D), jnp.float32),
    )(gx, gy2, wint)


def _edge_bwd_body(geo_ref, xj_ref, ge_ref, gdin_ref, wf1_ref, bf1_ref,
                   wf2_ref, bf2_ref, wf2t_ref, wf1t_ref, gxj_ref, gd_ref):
    geo = geo_ref[...]
    d = geo[:, 3:4]
    fcut = geo[:, 4:5]
    rbf, mu = _rbf_of(d)
    a = jnp.dot(rbf, wf1_ref[...],
                preferred_element_type=jnp.float32) + bf1_ref[...]
    f = jnp.dot(_ssp(a), wf2_ref[...],
                preferred_element_type=jnp.float32) + bf2_ref[...]
    ge = ge_ref[...]
    gxj_ref[...] = ge * (f * fcut)
    gw = ge * xj_ref[...]
    gf = gw * fcut
    gfc = jnp.sum(gw * f, axis=1, keepdims=True)
    ga = jnp.dot(gf, wf2t_ref[...],
                 preferred_element_type=jnp.float32) * jax.nn.sigmoid(a)
    grbf = jnp.dot(ga, wf1t_ref[...], preferred_element_type=jnp.float32)
    gd_rbf = jnp.sum(grbf * (-2.0 * GAMMA) * (d - mu) * rbf,
                     axis=1, keepdims=True)
    gd_ref[...] = gdin_ref[...] + gd_rbf + gfc * geo[:, 5:6]


def _tc_edge_bwd(geo, xj, ge, gdin, wf1, bf1, wf2, bf2, wf2t, wf1t):
    return pl.pallas_call(
        _edge_bwd_body,
        grid=(E // BE,),
        in_specs=[pl.BlockSpec((BE, 8), lambda i: (i, 0)),
                  pl.BlockSpec((BE, D), lambda i: (i, 0)),
                  pl.BlockSpec((BE, D), lambda i: (i, 0)),
                  pl.BlockSpec((BE, 1), lambda i: (i, 0)),
                  _full((NRBF, D)), _full((1, D)), _full((D, D)),
                  _full((1, D)), _full((D, D)), _full((D, NRBF))],
        out_specs=[pl.BlockSpec((BE, D), lambda i: (i, 0)),
                   pl.BlockSpec((BE, 1), lambda i: (i, 0))],
        out_shape=[jax.ShapeDtypeStruct((E, D), jnp.float32),
                   jax.ShapeDtypeStruct((E, 1), jnp.float32)],
    )(geo, xj, ge, gdin, wf1, bf1, wf2, bf2, wf2t, wf1t)


def _edge_final_body(geo_ref, gd_ref, grij_ref):
    geo = geo_ref[...]
    rij = geo[:, :3]
    d = geo[:, 3:4]
    s = gd_ref[...] / d
    pad = jnp.zeros((geo.shape[0], 5), jnp.float32)
    grij_ref[...] = jnp.concatenate([s * rij, pad], axis=1)


def _tc_edge_final(geo, gd):
    return pl.pallas_call(
        _edge_final_body,
        grid=(E // BE,),
        in_specs=[pl.BlockSpec((BE, 8), lambda i: (i, 0)),
                  pl.BlockSpec((BE, 1), lambda i: (i, 0))],
        out_specs=pl.BlockSpec((BE, 8), lambda i: (i, 0)),
        out_shape=jax.ShapeDtypeStruct((E, 8), jnp.float32),
    )(geo, gd)


def _combine_body(gi_ref, gj_ref, act_ref):
    g = gi_ref[0] + gi_ref[1] - gj_ref[0] - gj_ref[1]
    act_ref[...] = g[:, :3]


def _tc_combine(gi2, gj2):
    return pl.pallas_call(
        _combine_body,
        grid=(N // BN,),
        in_specs=[pl.BlockSpec((NC, BN, 8), lambda i: (0, i, 0)),
                  pl.BlockSpec((NC, BN, 8), lambda i: (0, i, 0))],
        out_specs=pl.BlockSpec((BN, 3), lambda i: (i, 0)),
        out_shape=jax.ShapeDtypeStruct((N, 3), jnp.float32),
    )(gi2, gj2)


def kernel(R, Z, idx_i, idx_j, emb, Wf1, bf1, Wf2, bf2, Win, Wout1, bout1,
           Wout2, bout2, Wa1, ba1, Wa2, ba2):
    idx_i3 = idx_i.astype(jnp.int32).reshape(NW, NCHUNK, CH)
    idx_j3 = idx_j.astype(jnp.int32).reshape(NW, NCHUNK, CH)
    z2 = Z.astype(jnp.int32).reshape(N, 1)
    rt = jnp.zeros((N, 16), jnp.float32).at[:, :3].set(R)

    bf1r = bf1.reshape(NINT, 1, D)
    bf2r = bf2.reshape(NINT, 1, D)
    bo1r = bout1.reshape(NINT, 1, D)
    bo2r = bout2.reshape(NINT, 1, D)
    ba1r = ba1.reshape(1, D // 2)
    ba2r = ba2.reshape(1, 1)
    wa2r = Wa2.reshape(1, D // 2)
    wa1t = jnp.transpose(Wa1)
    wf2t = jnp.transpose(Wf2, (0, 2, 1))
    wf1t = jnp.transpose(Wf1, (0, 2, 1))
    wo1t = jnp.transpose(Wout1, (0, 2, 1))
    wo2t = jnp.transpose(Wout2, (0, 2, 1))
    wint = jnp.transpose(Win, (0, 2, 1))

    ri, rj = _sc_gather2(rt, idx_i3, idx_j3)
    geo = _tc_geom(ri, rj)

    x, y = _tc_atom0(z2, emb, Win[0])
    ms, xjs = [], []
    for b in range(NINT):
        xj = _sc_gather(y, idx_j3)
        p = _tc_edge_fwd(geo, xj, Wf1[b], bf1r[b], Wf2[b], bf2r[b])
        m2 = _sc_scatter(p, idx_i3)
        winn = Win[b + 1] if b + 1 < NINT else Win[0]
        m, x, y = _tc_atom_fwd(m2, x, Wout1[b], bo1r[b], Wout2[b], bo2r[b],
                               winn, b + 1 < NINT)
        ms.append(m)
        xjs.append(xj)

    gx, e = _tc_head(x, Wa1, ba1r, wa2r, ba2r, wa1t)
    gd = jnp.zeros((E, 1), jnp.float32)
    for b in reversed(range(NINT)):
        gm = _tc_atom_bwd(gx, ms[b], Wout1[b], bo1r[b], wo2t[b], wo1t[b])
        ge = _sc_gather(gm, idx_i3)
        gxj, gd = _tc_edge_bwd(geo, xjs[b], ge, gd, Wf1[b], bf1r[b],
                               Wf2[b], bf2r[b], wf2t[b], wf1t[b])
        gy2 = _sc_scatter(gxj, idx_j3)
        gx = _tc_atom_acc(gx, gy2, wint[b])

    grij = _tc_edge_final(geo, gd)
    gi2, gj2 = _sc_scatter2(grij, idx_i3, idx_j3)
    action = _tc_combine(gi2, gj2)
    return (action, e[0, 0])


# speedup vs baseline: 1.4643x; 1.0126x over previous
import functools

import jax
import jax.numpy as jnp
from jax import lax
from jax.experimental import pallas as pl
from jax.experimental.pallas import tpu as pltpu
from jax.experimental.pallas import tpu_sc as plsc

N = 10000
E = 320000
D = 128
NRBF = 20
NINT = 3
CUTOFF = 5.0
GAMMA = 10.0
ZMAX = 100
LOG2 = 0.6931471805599453

NC, NS = 2, 16
NW = NC * NS
EPW = E // NW
CH = 40
NCHUNK = EPW // CH
CHG = 80
NCHUNKG = EPW // CHG
CW = 80
NROWCH = N // CW

BE = 2000
BN = 2000


def _ssp(x):
    return jax.nn.softplus(x) - LOG2


GNBUF = 5
SNBUF = 2


def _gather_pipeline(table, idx_v, out, base, rows, sem_g, sem_o):
    NBUF = GNBUF
    ROUNDS = NCHUNKG // NBUF
    for b in range(NBUF):
        pltpu.async_copy(table.at[idx_v.at[b]], rows.at[b], sem_g[b])

    def rnd(r, carry):
        for b in range(NBUF):
            g = r * NBUF + b
            off = base + g * CHG
            pltpu.make_async_copy(
                table.at[idx_v.at[g]], rows.at[b], sem_g[b]).wait()
            pltpu.async_copy(rows.at[b], out.at[pl.ds(off, CHG)], sem_o[b])
        for b in range(NBUF):
            g = r * NBUF + b
            g2 = g + NBUF

            @pl.when(g2 < NCHUNKG)
            def _():
                pltpu.make_async_copy(
                    rows.at[b], out.at[pl.ds(base + g * CHG, CHG)],
                    sem_o[b]).wait()
                pltpu.async_copy(table.at[idx_v.at[g2]], rows.at[b],
                                 sem_g[b])
        return carry

    lax.fori_loop(0, ROUNDS, rnd, 0)
    for b in range(NBUF):
        g = (ROUNDS - 1) * NBUF + b
        pltpu.make_async_copy(
            rows.at[b], out.at[pl.ds(base + g * CHG, CHG)], sem_o[b]).wait()


def _sc_gather_body(table, idx3, out, idx_v, rows, *sems):
    c = lax.axis_index("c")
    s = lax.axis_index("s")
    wid = s * NC + c
    pltpu.sync_copy(idx3.at[wid], idx_v)
    _gather_pipeline(table, idx_v, out, wid * EPW, rows,
                     sems[:GNBUF], sems[GNBUF:])


def _sc_gather(table, idx3):
    dt = table.shape[1]
    mesh = plsc.VectorSubcoreMesh(core_axis_name="c", subcore_axis_name="s")
    k = pl.kernel(
        _sc_gather_body,
        out_type=jax.ShapeDtypeStruct((E, dt), jnp.float32),
        mesh=mesh,
        scratch_types=[
            pltpu.VMEM((NCHUNKG, CHG), jnp.int32),
            pltpu.VMEM((GNBUF, CHG, dt), jnp.float32),
        ] + [pltpu.SemaphoreType.DMA] * (2 * GNBUF),
        compiler_params=pltpu.CompilerParams(
            use_tc_tiling_on_sc=(dt % 128 == 0)),
    )
    return k(table, idx3)


def _sc_gather2_body(table, idxa3, idxb3, outa, outb, idx_v, rows, *sems):
    c = lax.axis_index("c")
    s = lax.axis_index("s")
    wid = s * NC + c
    base = wid * EPW
    pltpu.sync_copy(idxa3.at[wid], idx_v)
    _gather_pipeline(table, idx_v, outa, base, rows, sems[:GNBUF],
                     sems[GNBUF:])
    pltpu.sync_copy(idxb3.at[wid], idx_v)
    _gather_pipeline(table, idx_v, outb, base, rows, sems[:GNBUF],
                     sems[GNBUF:])


def _sc_gather2(table, idxa3, idxb3):
    dt = table.shape[1]
    mesh = plsc.VectorSubcoreMesh(core_axis_name="c", subcore_axis_name="s")
    k = pl.kernel(
        _sc_gather2_body,
        out_type=[jax.ShapeDtypeStruct((E, dt), jnp.float32),
                  jax.ShapeDtypeStruct((E, dt), jnp.float32)],
        mesh=mesh,
        scratch_types=[
            pltpu.VMEM((NCHUNKG, CHG), jnp.int32),
            pltpu.VMEM((GNBUF, CHG, dt), jnp.float32),
        ] + [pltpu.SemaphoreType.DMA] * (2 * GNBUF),
        compiler_params=pltpu.CompilerParams(
            use_tc_tiling_on_sc=(dt % 128 == 0)),
    )
    return k(table, idxa3, idxb3)


def _acc_zero(s, zeros, acc_s):
    for t in range((NROWCH + NS - 1) // NS):
        k = s + NS * t

        @pl.when(k < NROWCH)
        def _():
            pltpu.sync_copy(zeros, acc_s.at[pl.ds(k * CW, CW)])


def _acc_writeout(c, s, acc_s, out):
    for t in range((NROWCH + NS - 1) // NS):
        k = s + NS * t

        @pl.when(k < NROWCH)
        def _():
            pltpu.sync_copy(acc_s.at[pl.ds(k * CW, CW)],
                            out.at[c, pl.ds(k * CW, CW)])


def _scatter_pipeline(vals, idx_vs, accs, base, vals_v, sem_v, sem_a):
    NBUF = SNBUF
    ROUNDS = NCHUNK // NBUF
    for b in range(NBUF):
        pltpu.async_copy(vals.at[pl.ds(base + b * CH, CH)], vals_v.at[b],
                         sem_v[b])

    def rnd(r, carry):
        descs = {}
        for b in range(NBUF):
            g = r * NBUF + b
            pltpu.make_async_copy(
                vals.at[pl.ds(base + g * CH, CH)], vals_v.at[b],
                sem_v[b]).wait()
            descs[b] = [
                pltpu.async_copy(vals_v.at[b], acc.at[iv.at[g]], sem_a[b],
                                 add=True)
                for acc, iv in zip(accs, idx_vs)]
        for b in range(NBUF):
            g2 = r * NBUF + b + NBUF

            @pl.when(g2 < NCHUNK)
            def _():
                for dsc in descs[b]:
                    dsc.wait()
                pltpu.async_copy(vals.at[pl.ds(base + g2 * CH, CH)],
                                 vals_v.at[b], sem_v[b])
        return carry

    lax.fori_loop(0, ROUNDS, rnd, 0)
    for b in range(NBUF):
        g = (ROUNDS - 1) * NBUF + b
        for acc, iv in zip(accs, idx_vs):
            pltpu.make_async_copy(vals_v.at[b], acc.at[iv.at[g]],
                                  sem_a[b]).wait()


def _sc_scatter_body(vals, idx3, zeros, out, idx_v, vals_v, acc_s, *sems):
    c = lax.axis_index("c")
    s = lax.axis_index("s")
    wid = s * NC + c
    pltpu.sync_copy(idx3.at[wid], idx_v)
    _acc_zero(s, zeros, acc_s)
    plsc.subcore_barrier()
    _scatter_pipeline(vals, [idx_v], [acc_s], wid * EPW, vals_v,
                      sems[:SNBUF], sems[SNBUF:])
    plsc.subcore_barrier()
    _acc_writeout(c, s, acc_s, out)


def _sc_scatter(vals, idx3):
    dt = vals.shape[1]
    zeros = jnp.zeros((CW, dt), jnp.float32)
    mesh = plsc.VectorSubcoreMesh(core_axis_name="c", subcore_axis_name="s")
    k = pl.kernel(
        _sc_scatter_body,
        out_type=jax.ShapeDtypeStruct((NC, N, dt), jnp.float32),
        mesh=mesh,
        scratch_types=[
            pltpu.VMEM((NCHUNK, CH), jnp.int32),
            pltpu.VMEM((SNBUF, CH, dt), jnp.float32),
            pltpu.VMEM_SHARED((N, dt), jnp.float32),
        ] + [pltpu.SemaphoreType.DMA] * (2 * SNBUF),
        compiler_params=pltpu.CompilerParams(
            use_tc_tiling_on_sc=(dt % 128 == 0)),
    )
    return k(vals, idx3, zeros)


def _sc_scatter2_body(vals, idxa3, idxb3, zeros, outa, outb, idxa_v, idxb_v,
                      vals_v, acca_s, accb_s, *sems):
    c = lax.axis_index("c")
    s = lax.axis_index("s")
    wid = s * NC + c
    pltpu.sync_copy(idxa3.at[wid], idxa_v)
    pltpu.sync_copy(idxb3.at[wid], idxb_v)
    _acc_zero(s, zeros, acca_s)
    _acc_zero(s, zeros, accb_s)
    plsc.subcore_barrier()
    _scatter_pipeline(vals, [idxa_v, idxb_v], [acca_s, accb_s], wid * EPW,
                      vals_v, sems[:SNBUF], sems[SNBUF:])
    plsc.subcore_barrier()
    _acc_writeout(c, s, acca_s, outa)
    _acc_writeout(c, s, accb_s, outb)


def _sc_scatter2(vals, idxa3, idxb3):
    dt = vals.shape[1]
    zeros = jnp.zeros((CW, dt), jnp.float32)
    mesh = plsc.VectorSubcoreMesh(core_axis_name="c", subcore_axis_name="s")
    k = pl.kernel(
        _sc_scatter2_body,
        out_type=[jax.ShapeDtypeStruct((NC, N, dt), jnp.float32),
                  jax.ShapeDtypeStruct((NC, N, dt), jnp.float32)],
        mesh=mesh,
        scratch_types=[
            pltpu.VMEM((NCHUNK, CH), jnp.int32),
            pltpu.VMEM((NCHUNK, CH), jnp.int32),
            pltpu.VMEM((SNBUF, CH, dt), jnp.float32),
            pltpu.VMEM_SHARED((N, dt), jnp.float32),
            pltpu.VMEM_SHARED((N, dt), jnp.float32),
        ] + [pltpu.SemaphoreType.DMA] * (2 * SNBUF),
        compiler_params=pltpu.CompilerParams(
            use_tc_tiling_on_sc=(dt % 128 == 0)),
    )
    return k(vals, idxa3, idxb3, zeros)


def _full(shape):
    return pl.BlockSpec(shape, lambda i: (0,) * len(shape))


BEG = 2000


def _geom_body(ri_ref, rj_ref, geo_ref):
    rij = rj_ref[:, :3] - ri_ref[:, :3]
    d2 = jnp.sum(rij * rij, axis=1, keepdims=True) + 1e-12
    d = jnp.sqrt(d2)
    mask = d < CUTOFF
    ang = jnp.pi * d / CUTOFF
    fcut = 0.5 * (jnp.cos(ang) + 1.0) * mask
    dfc = (-0.5 * jnp.pi / CUTOFF) * jnp.sin(ang) * mask
    pad = jnp.zeros((BEG, 2), jnp.float32)
    geo_ref[...] = jnp.concatenate([rij, d, fcut, dfc, pad], axis=1)


def _tc_geom(ri, rj):
    return pl.pallas_call(
        _geom_body,
        grid=(E // BEG,),
        in_specs=[pl.BlockSpec((BEG, 16), lambda i: (i, 0)),
                  pl.BlockSpec((BEG, 16), lambda i: (i, 0))],
        out_specs=pl.BlockSpec((BEG, 8), lambda i: (i, 0)),
        out_shape=jax.ShapeDtypeStruct((E, 8), jnp.float32),
    )(ri, rj)


def _atom0_body(z_ref, emb_ref, win_ref, x_ref, y_ref):
    z = z_ref[...]
    oh = (z == lax.broadcasted_iota(jnp.int32, (z.shape[0], ZMAX), 1))
    x = jnp.dot(oh.astype(jnp.float32), emb_ref[...],
                preferred_element_type=jnp.float32)
    x_ref[...] = x
    y_ref[...] = jnp.dot(x, win_ref[...], preferred_element_type=jnp.float32)


def _tc_atom0(z2, emb, win0):
    return pl.pallas_call(
        _atom0_body,
        grid=(N // BN,),
        in_specs=[pl.BlockSpec((BN, 1), lambda i: (i, 0)),
                  _full((ZMAX, D)), _full((D, D))],
        out_specs=[pl.BlockSpec((BN, D), lambda i: (i, 0)),
                   pl.BlockSpec((BN, D), lambda i: (i, 0))],
        out_shape=[jax.ShapeDtypeStruct((N, D), jnp.float32),
                   jax.ShapeDtypeStruct((N, D), jnp.float32)],
    )(z2, emb, win0)


def _rbf_of(d):
    mu = (CUTOFF / (NRBF - 1)) * lax.broadcasted_iota(
        jnp.int32, (1, NRBF), 1).astype(jnp.float32)
    return jnp.exp(-GAMMA * (d - mu) ** 2), mu


def _edge_fwd_body(geo_ref, xj_ref, wf1_ref, bf1_ref, wf2_ref, bf2_ref,
                   p_ref):
    geo = geo_ref[...]
    d = geo[:, 3:4]
    fcut = geo[:, 4:5]
    rbf, _ = _rbf_of(d)
    a = jnp.dot(rbf, wf1_ref[...],
                preferred_element_type=jnp.float32) + bf1_ref[...]
    f = jnp.dot(_ssp(a), wf2_ref[...],
                preferred_element_type=jnp.float32) + bf2_ref[...]
    p_ref[...] = xj_ref[...] * (f * fcut)


def _tc_edge_fwd(geo, xj, wf1, bf1, wf2, bf2):
    return pl.pallas_call(
        _edge_fwd_body,
        grid=(E // BE,),
        in_specs=[pl.BlockSpec((BE, 8), lambda i: (i, 0)),
                  pl.BlockSpec((BE, D), lambda i: (i, 0)),
                  _full((NRBF, D)), _full((1, D)), _full((D, D)),
                  _full((1, D))],
        out_specs=pl.BlockSpec((BE, D), lambda i: (i, 0)),
        out_shape=jax.ShapeDtypeStruct((E, D), jnp.float32),
    )(geo, xj, wf1, bf1, wf2, bf2)


def _atom_fwd_body(has_next, m2_ref, x_ref, w1_ref, b1_ref, w2_ref, b2_ref,
                   winn_ref, m_ref, xn_ref, yn_ref):
    m = m2_ref[0] + m2_ref[1]
    h = jnp.dot(m, w1_ref[...],
                preferred_element_type=jnp.float32) + b1_ref[...]
    v = jnp.dot(_ssp(h), w2_ref[...],
                preferred_element_type=jnp.float32) + b2_ref[...]
    xn = x_ref[...] + v
    m_ref[...] = m
    xn_ref[...] = xn
    if has_next:
        yn_ref[...] = jnp.dot(xn, winn_ref[...],
                              preferred_element_type=jnp.float32)
    else:
        yn_ref[...] = xn


def _tc_atom_fwd(m2, x, w1, b1, w2, b2, winn, has_next):
    return pl.pallas_call(
        functools.partial(_atom_fwd_body, has_next),
        grid=(N // BN,),
        in_specs=[pl.BlockSpec((NC, BN, D), lambda i: (0, i, 0)),
                  pl.BlockSpec((BN, D), lambda i: (i, 0)),
                  _full((D, D)), _full((1, D)), _full((D, D)),
                  _full((1, D)), _full((D, D))],
        out_specs=[pl.BlockSpec((BN, D), lambda i: (i, 0)),
                   pl.BlockSpec((BN, D), lambda i: (i, 0)),
                   pl.BlockSpec((BN, D), lambda i: (i, 0))],
        out_shape=[jax.ShapeDtypeStruct((N, D), jnp.float32),
                   jax.ShapeDtypeStruct((N, D), jnp.float32),
                   jax.ShapeDtypeStruct((N, D), jnp.float32)],
    )(m2, x, w1, b1, w2, b2, winn)


def _head_body(x3_ref, wa1_ref, ba1_ref, wa2r_ref, ba2_ref, wa1t_ref,
               gx_ref, e_ref):
    pi = pl.program_id(0)
    x3 = x3_ref[...]
    g = jnp.dot(x3, wa1_ref[...],
                preferred_element_type=jnp.float32) + ba1_ref[...]

    @pl.when(pi == 0)
    def _():
        e_ref[...] = jnp.zeros_like(e_ref)

    e_ref[...] += (jnp.sum(_ssp(g) * wa2r_ref[...], keepdims=True)
                   + x3.shape[0] * ba2_ref[...])
    gg = jax.nn.sigmoid(g) * wa2r_ref[...]
    gx_ref[...] = jnp.dot(gg, wa1t_ref[...],
                          preferred_element_type=jnp.float32)


def _tc_head(x3, wa1, ba1, wa2r, ba2, wa1t):
    return pl.pallas_call(
        _head_body,
        grid=(N // BN,),
        in_specs=[pl.BlockSpec((BN, D), lambda i: (i, 0)),
                  _full((D, D // 2)), _full((1, D // 2)),
                  _full((1, D // 2)), _full((1, 1)),
                  _full((D // 2, D))],
        out_specs=[pl.BlockSpec((BN, D), lambda i: (i, 0)),
                   pl.BlockSpec((1, 1), lambda i: (0, 0))],
        out_shape=[jax.ShapeDtypeStruct((N, D), jnp.float32),
                   jax.ShapeDtypeStruct((1, 1), jnp.float32)],
    )(x3, wa1, ba1, wa2r, ba2, wa1t)


def _atom_bwd_body(gx_ref, m_ref, w1_ref, b1_ref, w2t_ref, w1t_ref, gm_ref):
    h = jnp.dot(m_ref[...], w1_ref[...],
                preferred_element_type=jnp.float32) + b1_ref[...]
    gh = jnp.dot(gx_ref[...], w2t_ref[...],
                 preferred_element_type=jnp.float32) * jax.nn.sigmoid(h)
    gm_ref[...] = jnp.dot(gh, w1t_ref[...],
                          preferred_element_type=jnp.float32)


def _tc_atom_bwd(gx, m, w1, b1, w2t, w1t):
    return pl.pallas_call(
        _atom_bwd_body,
        grid=(N // BN,),
        in_specs=[pl.BlockSpec((BN, D), lambda i: (i, 0)),
                  pl.BlockSpec((BN, D), lambda i: (i, 0)),
                  _full((D, D)), _full((1, D)), _full((D, D)),
                  _full((D, D))],
        out_specs=pl.BlockSpec((BN, D), lambda i: (i, 0)),
        out_shape=jax.ShapeDtypeStruct((N, D), jnp.float32),
    )(gx, m, w1, b1, w2t, w1t)


def _atom_acc_body(gx_ref, gy2_ref, wint_ref, gxn_ref):
    gy = gy2_ref[0] + gy2_ref[1]
    gxn_ref[...] = gx_ref[...] + jnp.dot(
        gy, wint_ref[...], preferred_element_type=jnp.float32)


def _tc_atom_acc(gx, gy2, wint):
    return pl.pallas_call(
        _atom_acc_body,
        grid=(N // BN,),
        in_specs=[pl.BlockSpec((BN, D), lambda i: (i, 0)),
                  pl.BlockSpec((NC, BN, D), lambda i: (0, i, 0)),
                  _full((D, D))],
        out_specs=pl.BlockSpec((BN, D), lambda i: (i, 0)),
        out_shape=jax.ShapeDtypeStruct((N, D), jnp.float32),
    )(gx, gy2, wint)


def _edge_bwd_body(geo_ref, xj_ref, ge_ref, gdin_ref, wf1_ref, bf1_ref,
                   wf2_ref, bf2_ref, wf2t_ref, wf1t_ref, gxj_ref, gd_ref):
    geo = geo_ref[...]
    d = geo[:, 3:4]
    fcut = geo[:, 4:5]
    rbf, mu = _rbf_of(d)
    a = jnp.dot(rbf, wf1_ref[...],
                preferred_element_type=jnp.float32) + bf1_ref[...]
    f = jnp.dot(_ssp(a), wf2_ref[...],
                preferred_element_type=jnp.float32) + bf2_ref[...]
    ge = ge_ref[...]
    gxj_ref[...] = ge * (f * fcut)
    gw = ge * xj_ref[...]
    gf = gw * fcut
    gfc = jnp.sum(gw * f, axis=1, keepdims=True)
    ga = jnp.dot(gf, wf2t_ref[...],
                 preferred_element_type=jnp.float32) * jax.nn.sigmoid(a)
    grbf = jnp.dot(ga, wf1t_ref[...], preferred_element_type=jnp.float32)
    gd_rbf = jnp.sum(grbf * (-2.0 * GAMMA) * (d - mu) * rbf,
                     axis=1, keepdims=True)
    gd_ref[...] = gdin_ref[...] + gd_rbf + gfc * geo[:, 5:6]


def _tc_edge_bwd(geo, xj, ge, gdin, wf1, bf1, wf2, bf2, wf2t, wf1t):
    return pl.pallas_call(
        _edge_bwd_body,
        grid=(E // BE,),
        in_specs=[pl.BlockSpec((BE, 8), lambda i: (i, 0)),
                  pl.BlockSpec((BE, D), lambda i: (i, 0)),
                  pl.BlockSpec((BE, D), lambda i: (i, 0)),
                  pl.BlockSpec((BE, 1), lambda i: (i, 0)),
                  _full((NRBF, D)), _full((1, D)), _full((D, D)),
                  _full((1, D)), _full((D, D)), _full((D, NRBF))],
        out_specs=[pl.BlockSpec((BE, D), lambda i: (i, 0)),
                   pl.BlockSpec((BE, 1), lambda i: (i, 0))],
        out_shape=[jax.ShapeDtypeStruct((E, D), jnp.float32),
                   jax.ShapeDtypeStruct((E, 1), jnp.float32)],
    )(geo, xj, ge, gdin, wf1, bf1, wf2, bf2, wf2t, wf1t)


def _edge_final_body(geo_ref, gd_ref, grij_ref):
    geo = geo_ref[...]
    rij = geo[:, :3]
    d = geo[:, 3:4]
    s = gd_ref[...] / d
    pad = jnp.zeros((geo.shape[0], 5), jnp.float32)
    grij_ref[...] = jnp.concatenate([s * rij, pad], axis=1)


def _tc_edge_final(geo, gd):
    return pl.pallas_call(
        _edge_final_body,
        grid=(E // BE,),
        in_specs=[pl.BlockSpec((BE, 8), lambda i: (i, 0)),
                  pl.BlockSpec((BE, 1), lambda i: (i, 0))],
        out_specs=pl.BlockSpec((BE, 8), lambda i: (i, 0)),
        out_shape=jax.ShapeDtypeStruct((E, 8), jnp.float32),
    )(geo, gd)


def _combine_body(gi_ref, gj_ref, act_ref):
    g = gi_ref[0] + gi_ref[1] - gj_ref[0] - gj_ref[1]
    act_ref[...] = g[:, :3]


def _tc_combine(gi2, gj2):
    return pl.pallas_call(
        _combine_body,
        grid=(N // BN,),
        in_specs=[pl.BlockSpec((NC, BN, 8), lambda i: (0, i, 0)),
                  pl.BlockSpec((NC, BN, 8), lambda i: (0, i, 0))],
        out_specs=pl.BlockSpec((BN, 3), lambda i: (i, 0)),
        out_shape=jax.ShapeDtypeStruct((N, 3), jnp.float32),
    )(gi2, gj2)


def kernel(R, Z, idx_i, idx_j, emb, Wf1, bf1, Wf2, bf2, Win, Wout1, bout1,
           Wout2, bout2, Wa1, ba1, Wa2, ba2):
    idx_i3 = idx_i.astype(jnp.int32).reshape(NW, NCHUNK, CH)
    idx_j3 = idx_j.astype(jnp.int32).reshape(NW, NCHUNK, CH)
    idx_i3g = idx_i.astype(jnp.int32).reshape(NW, NCHUNKG, CHG)
    idx_j3g = idx_j.astype(jnp.int32).reshape(NW, NCHUNKG, CHG)
    z2 = Z.astype(jnp.int32).reshape(N, 1)
    rt = jnp.zeros((N, 16), jnp.float32).at[:, :3].set(R)

    bf1r = bf1.reshape(NINT, 1, D)
    bf2r = bf2.reshape(NINT, 1, D)
    bo1r = bout1.reshape(NINT, 1, D)
    bo2r = bout2.reshape(NINT, 1, D)
    ba1r = ba1.reshape(1, D // 2)
    ba2r = ba2.reshape(1, 1)
    wa2r = Wa2.reshape(1, D // 2)
    wa1t = jnp.transpose(Wa1)
    wf2t = jnp.transpose(Wf2, (0, 2, 1))
    wf1t = jnp.transpose(Wf1, (0, 2, 1))
    wo1t = jnp.transpose(Wout1, (0, 2, 1))
    wo2t = jnp.transpose(Wout2, (0, 2, 1))
    wint = jnp.transpose(Win, (0, 2, 1))

    ri, rj = _sc_gather2(rt, idx_i3g, idx_j3g)
    geo = _tc_geom(ri, rj)

    x, y = _tc_atom0(z2, emb, Win[0])
    ms, xjs = [], []
    for b in range(NINT):
        xj = _sc_gather(y, idx_j3g)
        p = _tc_edge_fwd(geo, xj, Wf1[b], bf1r[b], Wf2[b], bf2r[b])
        m2 = _sc_scatter(p, idx_i3)
        winn = Win[b + 1] if b + 1 < NINT else Win[0]
        m, x, y = _tc_atom_fwd(m2, x, Wout1[b], bo1r[b], Wout2[b], bo2r[b],
                               winn, b + 1 < NINT)
        ms.append(m)
        xjs.append(xj)

    gx, e = _tc_head(x, Wa1, ba1r, wa2r, ba2r, wa1t)
    gd = jnp.zeros((E, 1), jnp.float32)
    for b in reversed(range(NINT)):
        gm = _tc_atom_bwd(gx, ms[b], Wout1[b], bo1r[b], wo2t[b], wo1t[b])
        ge = _sc_gather(gm, idx_i3g)
        gxj, gd = _tc_edge_bwd(geo, xjs[b], ge, gd, Wf1[b], bf1r[b],
                               Wf2[b], bf2r[b], wf2t[b], wf1t[b])
        gy2 = _sc_scatter(gxj, idx_j3)
        gx = _tc_atom_acc(gx, gy2, wint[b])

    grij = _tc_edge_final(geo, gd)
    gi2, gj2 = _sc_scatter2(grij, idx_i3, idx_j3)
    action = _tc_combine(gi2, gj2)
    return (action, e[0, 0])


# speedup vs baseline: 1.4919x; 1.0189x over previous
import functools

import jax
import jax.numpy as jnp
from jax import lax
from jax.experimental import pallas as pl
from jax.experimental.pallas import tpu as pltpu
from jax.experimental.pallas import tpu_sc as plsc

N = 10000
E = 320000
D = 128
NRBF = 20
NINT = 3
CUTOFF = 5.0
GAMMA = 10.0
ZMAX = 100
LOG2 = 0.6931471805599453

NC, NS = 2, 16
NW = NC * NS
EPW = E // NW
CH = 40
NCHUNK = EPW // CH
CHG = 80
NCHUNKG = EPW // CHG
CW = 80
NROWCH = N // CW

BE = 2000
BN = 2000


def _ssp(x):
    return jax.nn.softplus(x) - LOG2


GNBUF = 5
SNBUF = 2


def _gather_pipeline(table, idx_v, out, base, rows, sem_g, sem_o):
    NBUF = GNBUF
    ROUNDS = NCHUNKG // NBUF
    for b in range(NBUF):
        pltpu.async_copy(table.at[idx_v.at[b]], rows.at[b], sem_g[b])

    def rnd(r, carry):
        for b in range(NBUF):
            g = r * NBUF + b
            off = base + g * CHG
            pltpu.make_async_copy(
                table.at[idx_v.at[g]], rows.at[b], sem_g[b]).wait()
            pltpu.async_copy(rows.at[b], out.at[pl.ds(off, CHG)], sem_o[b])
        for b in range(NBUF):
            g = r * NBUF + b
            g2 = g + NBUF

            @pl.when(g2 < NCHUNKG)
            def _():
                pltpu.make_async_copy(
                    rows.at[b], out.at[pl.ds(base + g * CHG, CHG)],
                    sem_o[b]).wait()
                pltpu.async_copy(table.at[idx_v.at[g2]], rows.at[b],
                                 sem_g[b])
        return carry

    lax.fori_loop(0, ROUNDS, rnd, 0)
    for b in range(NBUF):
        g = (ROUNDS - 1) * NBUF + b
        pltpu.make_async_copy(
            rows.at[b], out.at[pl.ds(base + g * CHG, CHG)], sem_o[b]).wait()


def _sc_gather_body(table, idx3, out, idx_v, rows, *sems):
    c = lax.axis_index("c")
    s = lax.axis_index("s")
    wid = s * NC + c
    pltpu.sync_copy(idx3.at[wid], idx_v)
    _gather_pipeline(table, idx_v, out, wid * EPW, rows,
                     sems[:GNBUF], sems[GNBUF:])


def _sc_gather(table, idx3):
    dt = table.shape[1]
    mesh = plsc.VectorSubcoreMesh(core_axis_name="c", subcore_axis_name="s")
    k = pl.kernel(
        _sc_gather_body,
        out_type=jax.ShapeDtypeStruct((E, dt), jnp.float32),
        mesh=mesh,
        scratch_types=[
            pltpu.VMEM((NCHUNKG, CHG), jnp.int32),
            pltpu.VMEM((GNBUF, CHG, dt), jnp.float32),
        ] + [pltpu.SemaphoreType.DMA] * (2 * GNBUF),
        compiler_params=pltpu.CompilerParams(
            use_tc_tiling_on_sc=(dt % 128 == 0)),
    )
    return k(table, idx3)


def _acc_zero(s, zeros, acc_s):
    for t in range((NROWCH + NS - 1) // NS):
        k = s + NS * t

        @pl.when(k < NROWCH)
        def _():
            pltpu.sync_copy(zeros, acc_s.at[pl.ds(k * CW, CW)])


def _acc_writeout(c, s, acc_s, out):
    for t in range((NROWCH + NS - 1) // NS):
        k = s + NS * t

        @pl.when(k < NROWCH)
        def _():
            pltpu.sync_copy(acc_s.at[pl.ds(k * CW, CW)],
                            out.at[c, pl.ds(k * CW, CW)])


def _scatter_pipeline(vals, idx_vs, accs, base, vals_v, sem_v, sem_a):
    NBUF = SNBUF
    ROUNDS = NCHUNK // NBUF
    for b in range(NBUF):
        pltpu.async_copy(vals.at[pl.ds(base + b * CH, CH)], vals_v.at[b],
                         sem_v[b])

    def rnd(r, carry):
        descs = {}
        for b in range(NBUF):
            g = r * NBUF + b
            pltpu.make_async_copy(
                vals.at[pl.ds(base + g * CH, CH)], vals_v.at[b],
                sem_v[b]).wait()
            descs[b] = [
                pltpu.async_copy(vals_v.at[b], acc.at[iv.at[g]], sem_a[b],
                                 add=True)
                for acc, iv in zip(accs, idx_vs)]
        for b in range(NBUF):
            g2 = r * NBUF + b + NBUF

            @pl.when(g2 < NCHUNK)
            def _():
                for dsc in descs[b]:
                    dsc.wait()
                pltpu.async_copy(vals.at[pl.ds(base + g2 * CH, CH)],
                                 vals_v.at[b], sem_v[b])
        return carry

    lax.fori_loop(0, ROUNDS, rnd, 0)
    for b in range(NBUF):
        g = (ROUNDS - 1) * NBUF + b
        for acc, iv in zip(accs, idx_vs):
            pltpu.make_async_copy(vals_v.at[b], acc.at[iv.at[g]],
                                  sem_a[b]).wait()


def _sc_scatter_body(vals, idx3, zeros, out, idx_v, vals_v, acc_s, *sems):
    c = lax.axis_index("c")
    s = lax.axis_index("s")
    wid = s * NC + c
    pltpu.sync_copy(idx3.at[wid], idx_v)
    _acc_zero(s, zeros, acc_s)
    plsc.subcore_barrier()
    _scatter_pipeline(vals, [idx_v], [acc_s], wid * EPW, vals_v,
                      sems[:SNBUF], sems[SNBUF:])
    plsc.subcore_barrier()
    _acc_writeout(c, s, acc_s, out)


def _sc_scatter(vals, idx3):
    dt = vals.shape[1]
    zeros = jnp.zeros((CW, dt), jnp.float32)
    mesh = plsc.VectorSubcoreMesh(core_axis_name="c", subcore_axis_name="s")
    k = pl.kernel(
        _sc_scatter_body,
        out_type=jax.ShapeDtypeStruct((NC, N, dt), jnp.float32),
        mesh=mesh,
        scratch_types=[
            pltpu.VMEM((NCHUNK, CH), jnp.int32),
            pltpu.VMEM((SNBUF, CH, dt), jnp.float32),
            pltpu.VMEM_SHARED((N, dt), jnp.float32),
        ] + [pltpu.SemaphoreType.DMA] * (2 * SNBUF),
        compiler_params=pltpu.CompilerParams(
            use_tc_tiling_on_sc=(dt % 128 == 0)),
    )
    return k(vals, idx3, zeros)


def _sc_scatter2_body(vals, idxa3, idxb3, zeros, outa, outb, idxa_v, idxb_v,
                      vals_v, acca_s, accb_s, *sems):
    c = lax.axis_index("c")
    s = lax.axis_index("s")
    wid = s * NC + c
    pltpu.sync_copy(idxa3.at[wid], idxa_v)
    pltpu.sync_copy(idxb3.at[wid], idxb_v)
    _acc_zero(s, zeros, acca_s)
    _acc_zero(s, zeros, accb_s)
    plsc.subcore_barrier()
    _scatter_pipeline(vals, [idxa_v, idxb_v], [acca_s, accb_s], wid * EPW,
                      vals_v, sems[:SNBUF], sems[SNBUF:])
    plsc.subcore_barrier()
    _acc_writeout(c, s, acca_s, outa)
    _acc_writeout(c, s, accb_s, outb)


def _sc_scatter2(vals, idxa3, idxb3):
    dt = vals.shape[1]
    zeros = jnp.zeros((CW, dt), jnp.float32)
    mesh = plsc.VectorSubcoreMesh(core_axis_name="c", subcore_axis_name="s")
    k = pl.kernel(
        _sc_scatter2_body,
        out_type=[jax.ShapeDtypeStruct((NC, N, dt), jnp.float32),
                  jax.ShapeDtypeStruct((NC, N, dt), jnp.float32)],
        mesh=mesh,
        scratch_types=[
            pltpu.VMEM((NCHUNK, CH), jnp.int32),
            pltpu.VMEM((NCHUNK, CH), jnp.int32),
            pltpu.VMEM((SNBUF, CH, dt), jnp.float32),
            pltpu.VMEM_SHARED((N, dt), jnp.float32),
            pltpu.VMEM_SHARED((N, dt), jnp.float32),
        ] + [pltpu.SemaphoreType.DMA] * (2 * SNBUF),
        compiler_params=pltpu.CompilerParams(
            use_tc_tiling_on_sc=(dt % 128 == 0)),
    )
    return k(vals, idxa3, idxb3, zeros)


def _full(shape):
    return pl.BlockSpec(shape, lambda i: (0,) * len(shape))


BEG = 2000


def _geom_body(ri_ref, rj_ref, geo_ref):
    rij = rj_ref[:, :3] - ri_ref[:, :3]
    d2 = jnp.sum(rij * rij, axis=1, keepdims=True) + 1e-12
    d = jnp.sqrt(d2)
    mask = d < CUTOFF
    ang = jnp.pi * d / CUTOFF
    fcut = 0.5 * (jnp.cos(ang) + 1.0) * mask
    dfc = (-0.5 * jnp.pi / CUTOFF) * jnp.sin(ang) * mask
    pad = jnp.zeros((BEG, 2), jnp.float32)
    geo_ref[...] = jnp.concatenate([rij, d, fcut, dfc, pad], axis=1)


def _tc_geom(ri, rj):
    return pl.pallas_call(
        _geom_body,
        grid=(E // BEG,),
        in_specs=[pl.BlockSpec((BEG, D), lambda i: (i, 0)),
                  pl.BlockSpec((BEG, D), lambda i: (i, 0))],
        out_specs=pl.BlockSpec((BEG, 8), lambda i: (i, 0)),
        out_shape=jax.ShapeDtypeStruct((E, 8), jnp.float32),
    )(ri, rj)


def _atom0_body(z_ref, emb_ref, win_ref, x_ref, y_ref):
    z = z_ref[...]
    oh = (z == lax.broadcasted_iota(jnp.int32, (z.shape[0], ZMAX), 1))
    x = jnp.dot(oh.astype(jnp.float32), emb_ref[...],
                preferred_element_type=jnp.float32)
    x_ref[...] = x
    y_ref[...] = jnp.dot(x, win_ref[...], preferred_element_type=jnp.float32)


def _tc_atom0(z2, emb, win0):
    return pl.pallas_call(
        _atom0_body,
        grid=(N // BN,),
        in_specs=[pl.BlockSpec((BN, 1), lambda i: (i, 0)),
                  _full((ZMAX, D)), _full((D, D))],
        out_specs=[pl.BlockSpec((BN, D), lambda i: (i, 0)),
                   pl.BlockSpec((BN, D), lambda i: (i, 0))],
        out_shape=[jax.ShapeDtypeStruct((N, D), jnp.float32),
                   jax.ShapeDtypeStruct((N, D), jnp.float32)],
    )(z2, emb, win0)


def _rbf_of(d):
    mu = (CUTOFF / (NRBF - 1)) * lax.broadcasted_iota(
        jnp.int32, (1, NRBF), 1).astype(jnp.float32)
    return jnp.exp(-GAMMA * (d - mu) ** 2), mu


def _edge_fwd_body(geo_ref, xj_ref, wf1_ref, bf1_ref, wf2_ref, bf2_ref,
                   p_ref):
    geo = geo_ref[...]
    d = geo[:, 3:4]
    fcut = geo[:, 4:5]
    rbf, _ = _rbf_of(d)
    a = jnp.dot(rbf, wf1_ref[...],
                preferred_element_type=jnp.float32) + bf1_ref[...]
    f = jnp.dot(_ssp(a), wf2_ref[...],
                preferred_element_type=jnp.float32) + bf2_ref[...]
    p_ref[...] = xj_ref[...] * (f * fcut)


def _tc_edge_fwd(geo, xj, wf1, bf1, wf2, bf2):
    return pl.pallas_call(
        _edge_fwd_body,
        grid=(E // BE,),
        in_specs=[pl.BlockSpec((BE, 8), lambda i: (i, 0)),
                  pl.BlockSpec((BE, D), lambda i: (i, 0)),
                  _full((NRBF, D)), _full((1, D)), _full((D, D)),
                  _full((1, D))],
        out_specs=pl.BlockSpec((BE, D), lambda i: (i, 0)),
        out_shape=jax.ShapeDtypeStruct((E, D), jnp.float32),
    )(geo, xj, wf1, bf1, wf2, bf2)


def _atom_fwd_body(has_next, m2_ref, x_ref, w1_ref, b1_ref, w2_ref, b2_ref,
                   winn_ref, m_ref, xn_ref, yn_ref):
    m = m2_ref[0] + m2_ref[1]
    h = jnp.dot(m, w1_ref[...],
                preferred_element_type=jnp.float32) + b1_ref[...]
    v = jnp.dot(_ssp(h), w2_ref[...],
                preferred_element_type=jnp.float32) + b2_ref[...]
    xn = x_ref[...] + v
    m_ref[...] = m
    xn_ref[...] = xn
    if has_next:
        yn_ref[...] = jnp.dot(xn, winn_ref[...],
                              preferred_element_type=jnp.float32)
    else:
        yn_ref[...] = xn


def _tc_atom_fwd(m2, x, w1, b1, w2, b2, winn, has_next):
    return pl.pallas_call(
        functools.partial(_atom_fwd_body, has_next),
        grid=(N // BN,),
        in_specs=[pl.BlockSpec((NC, BN, D), lambda i: (0, i, 0)),
                  pl.BlockSpec((BN, D), lambda i: (i, 0)),
                  _full((D, D)), _full((1, D)), _full((D, D)),
                  _full((1, D)), _full((D, D))],
        out_specs=[pl.BlockSpec((BN, D), lambda i: (i, 0)),
                   pl.BlockSpec((BN, D), lambda i: (i, 0)),
                   pl.BlockSpec((BN, D), lambda i: (i, 0))],
        out_shape=[jax.ShapeDtypeStruct((N, D), jnp.float32),
                   jax.ShapeDtypeStruct((N, D), jnp.float32),
                   jax.ShapeDtypeStruct((N, D), jnp.float32)],
    )(m2, x, w1, b1, w2, b2, winn)


def _head_body(x3_ref, wa1_ref, ba1_ref, wa2r_ref, ba2_ref, wa1t_ref,
               gx_ref, e_ref):
    pi = pl.program_id(0)
    x3 = x3_ref[...]
    g = jnp.dot(x3, wa1_ref[...],
                preferred_element_type=jnp.float32) + ba1_ref[...]

    @pl.when(pi == 0)
    def _():
        e_ref[...] = jnp.zeros_like(e_ref)

    e_ref[...] += (jnp.sum(_ssp(g) * wa2r_ref[...], keepdims=True)
                   + x3.shape[0] * ba2_ref[...])
    gg = jax.nn.sigmoid(g) * wa2r_ref[...]
    gx_ref[...] = jnp.dot(gg, wa1t_ref[...],
                          preferred_element_type=jnp.float32)


def _tc_head(x3, wa1, ba1, wa2r, ba2, wa1t):
    return pl.pallas_call(
        _head_body,
        grid=(N // BN,),
        in_specs=[pl.BlockSpec((BN, D), lambda i: (i, 0)),
                  _full((D, D // 2)), _full((1, D // 2)),
                  _full((1, D // 2)), _full((1, 1)),
                  _full((D // 2, D))],
        out_specs=[pl.BlockSpec((BN, D), lambda i: (i, 0)),
                   pl.BlockSpec((1, 1), lambda i: (0, 0))],
        out_shape=[jax.ShapeDtypeStruct((N, D), jnp.float32),
                   jax.ShapeDtypeStruct((1, 1), jnp.float32)],
    )(x3, wa1, ba1, wa2r, ba2, wa1t)


def _atom_bwd_body(gx_ref, m_ref, w1_ref, b1_ref, w2t_ref, w1t_ref, gm_ref):
    h = jnp.dot(m_ref[...], w1_ref[...],
                preferred_element_type=jnp.float32) + b1_ref[...]
    gh = jnp.dot(gx_ref[...], w2t_ref[...],
                 preferred_element_type=jnp.float32) * jax.nn.sigmoid(h)
    gm_ref[...] = jnp.dot(gh, w1t_ref[...],
                          preferred_element_type=jnp.float32)


def _tc_atom_bwd(gx, m, w1, b1, w2t, w1t):
    return pl.pallas_call(
        _atom_bwd_body,
        grid=(N // BN,),
        in_specs=[pl.BlockSpec((BN, D), lambda i: (i, 0)),
                  pl.BlockSpec((BN, D), lambda i: (i, 0)),
                  _full((D, D)), _full((1, D)), _full((D, D)),
                  _full((D, D))],
        out_specs=pl.BlockSpec((BN, D), lambda i: (i, 0)),
        out_shape=jax.ShapeDtypeStruct((N, D), jnp.float32),
    )(gx, m, w1, b1, w2t, w1t)


def _atom_acc_body(gx_ref, gy2_ref, wint_ref, gxn_ref):
    gy = gy2_ref[0] + gy2_ref[1]
    gxn_ref[...] = gx_ref[...] + jnp.dot(
        gy, wint_ref[...], preferred_element_type=jnp.float32)


def _tc_atom_acc(gx, gy2, wint):
    return pl.pallas_call(
        _atom_acc_body,
        grid=(N // BN,),
        in_specs=[pl.BlockSpec((BN, D), lambda i: (i, 0)),
                  pl.BlockSpec((NC, BN, D), lambda i: (0, i, 0)),
                  _full((D, D))],
        out_specs=pl.BlockSpec((BN, D), lambda i: (i, 0)),
        out_shape=jax.ShapeDtypeStruct((N, D), jnp.float32),
    )(gx, gy2, wint)


def _edge_bwd_body(geo_ref, xj_ref, ge_ref, gdin_ref, wf1_ref, bf1_ref,
                   wf2_ref, bf2_ref, wf2t_ref, wf1t_ref, gxj_ref, gd_ref):
    geo = geo_ref[...]
    d = geo[:, 3:4]
    fcut = geo[:, 4:5]
    rbf, mu = _rbf_of(d)
    a = jnp.dot(rbf, wf1_ref[...],
                preferred_element_type=jnp.float32) + bf1_ref[...]
    f = jnp.dot(_ssp(a), wf2_ref[...],
                preferred_element_type=jnp.float32) + bf2_ref[...]
    ge = ge_ref[...]
    gxj_ref[...] = ge * (f * fcut)
    gw = ge * xj_ref[...]
    gf = gw * fcut
    gfc = jnp.sum(gw * f, axis=1, keepdims=True)
    ga = jnp.dot(gf, wf2t_ref[...],
                 preferred_element_type=jnp.float32) * jax.nn.sigmoid(a)
    grbf = jnp.dot(ga, wf1t_ref[...], preferred_element_type=jnp.float32)
    gd_rbf = jnp.sum(grbf * (-2.0 * GAMMA) * (d - mu) * rbf,
                     axis=1, keepdims=True)
    gd_ref[...] = gdin_ref[...] + gd_rbf + gfc * geo[:, 5:6]


def _tc_edge_bwd(geo, xj, ge, gdin, wf1, bf1, wf2, bf2, wf2t, wf1t):
    return pl.pallas_call(
        _edge_bwd_body,
        grid=(E // BE,),
        in_specs=[pl.BlockSpec((BE, 8), lambda i: (i, 0)),
                  pl.BlockSpec((BE, D), lambda i: (i, 0)),
                  pl.BlockSpec((BE, D), lambda i: (i, 0)),
                  pl.BlockSpec((BE, 1), lambda i: (i, 0)),
                  _full((NRBF, D)), _full((1, D)), _full((D, D)),
                  _full((1, D)), _full((D, D)), _full((D, NRBF))],
        out_specs=[pl.BlockSpec((BE, D), lambda i: (i, 0)),
                   pl.BlockSpec((BE, 1), lambda i: (i, 0))],
        out_shape=[jax.ShapeDtypeStruct((E, D), jnp.float32),
                   jax.ShapeDtypeStruct((E, 1), jnp.float32)],
    )(geo, xj, ge, gdin, wf1, bf1, wf2, bf2, wf2t, wf1t)


def _edge_final_body(geo_ref, gd_ref, grij_ref):
    geo = geo_ref[...]
    rij = geo[:, :3]
    d = geo[:, 3:4]
    s = gd_ref[...] / d
    pad = jnp.zeros((geo.shape[0], 5), jnp.float32)
    grij_ref[...] = jnp.concatenate([s * rij, pad], axis=1)


def _tc_edge_final(geo, gd):
    return pl.pallas_call(
        _edge_final_body,
        grid=(E // BE,),
        in_specs=[pl.BlockSpec((BE, 8), lambda i: (i, 0)),
                  pl.BlockSpec((BE, 1), lambda i: (i, 0))],
        out_specs=pl.BlockSpec((BE, 8), lambda i: (i, 0)),
        out_shape=jax.ShapeDtypeStruct((E, 8), jnp.float32),
    )(geo, gd)


def _combine_body(gi_ref, gj_ref, act_ref):
    g = gi_ref[0] + gi_ref[1] - gj_ref[0] - gj_ref[1]
    act_ref[...] = g[:, :3]


def _tc_combine(gi2, gj2):
    return pl.pallas_call(
        _combine_body,
        grid=(N // BN,),
        in_specs=[pl.BlockSpec((NC, BN, 8), lambda i: (0, i, 0)),
                  pl.BlockSpec((NC, BN, 8), lambda i: (0, i, 0))],
        out_specs=pl.BlockSpec((BN, 3), lambda i: (i, 0)),
        out_shape=jax.ShapeDtypeStruct((N, 3), jnp.float32),
    )(gi2, gj2)


def kernel(R, Z, idx_i, idx_j, emb, Wf1, bf1, Wf2, bf2, Win, Wout1, bout1,
           Wout2, bout2, Wa1, ba1, Wa2, ba2):
    idx_i3 = idx_i.astype(jnp.int32).reshape(NW, NCHUNK, CH)
    idx_j3 = idx_j.astype(jnp.int32).reshape(NW, NCHUNK, CH)
    idx_i3g = idx_i.astype(jnp.int32).reshape(NW, NCHUNKG, CHG)
    idx_j3g = idx_j.astype(jnp.int32).reshape(NW, NCHUNKG, CHG)
    z2 = Z.astype(jnp.int32).reshape(N, 1)
    rt = jnp.zeros((N, D), jnp.float32).at[:, :3].set(R)

    bf1r = bf1.reshape(NINT, 1, D)
    bf2r = bf2.reshape(NINT, 1, D)
    bo1r = bout1.reshape(NINT, 1, D)
    bo2r = bout2.reshape(NINT, 1, D)
    ba1r = ba1.reshape(1, D // 2)
    ba2r = ba2.reshape(1, 1)
    wa2r = Wa2.reshape(1, D // 2)
    wa1t = jnp.transpose(Wa1)
    wf2t = jnp.transpose(Wf2, (0, 2, 1))
    wf1t = jnp.transpose(Wf1, (0, 2, 1))
    wo1t = jnp.transpose(Wout1, (0, 2, 1))
    wo2t = jnp.transpose(Wout2, (0, 2, 1))
    wint = jnp.transpose(Win, (0, 2, 1))

    ri = _sc_gather(rt, idx_i3g)
    rj = _sc_gather(rt, idx_j3g)
    geo = _tc_geom(ri, rj)

    x, y = _tc_atom0(z2, emb, Win[0])
    ms, xjs = [], []
    for b in range(NINT):
        xj = _sc_gather(y, idx_j3g)
        p = _tc_edge_fwd(geo, xj, Wf1[b], bf1r[b], Wf2[b], bf2r[b])
        m2 = _sc_scatter(p, idx_i3)
        winn = Win[b + 1] if b + 1 < NINT else Win[0]
        m, x, y = _tc_atom_fwd(m2, x, Wout1[b], bo1r[b], Wout2[b], bo2r[b],
                               winn, b + 1 < NINT)
        ms.append(m)
        xjs.append(xj)

    gx, e = _tc_head(x, Wa1, ba1r, wa2r, ba2r, wa1t)
    gd = jnp.zeros((E, 1), jnp.float32)
    for b in reversed(range(NINT)):
        gm = _tc_atom_bwd(gx, ms[b], Wout1[b], bo1r[b], wo2t[b], wo1t[b])
        ge = _sc_gather(gm, idx_i3g)
        gxj, gd = _tc_edge_bwd(geo, xjs[b], ge, gd, Wf1[b], bf1r[b],
                               Wf2[b], bf2r[b], wf2t[b], wf1t[b])
        gy2 = _sc_scatter(gxj, idx_j3)
        gx = _tc_atom_acc(gx, gy2, wint[b])

    grij = _tc_edge_final(geo, gd)
    gi2, gj2 = _sc_scatter2(grij, idx_i3, idx_j3)
    action = _tc_combine(gi2, gj2)
    return (action, e[0, 0])
